# baseline (speedup 1.0000x reference)
import functools

import jax
import jax.numpy as jnp
from jax import lax
from jax.experimental import pallas as pl
from jax.experimental.pallas import tpu as pltpu
from jax.experimental.pallas import tpu_sc as plsc

N = 10000
E = 320000
D = 128
H = 64
C = 4
R = 3
G = 32

NW = 32
CHUNK = 128
NCH = 80
EPAD = NW * NCH * CHUNK
NACC = N + 16
TW = 128
DEGCOL = H

BN = 2000
NB = N // BN


def _dense1_body(x_ref, w_ref, y_ref):
    y = jax.lax.dot_general(
        x_ref[...], w_ref[0], (((1,), (0,)), ((), ())),
        preferred_element_type=jnp.float32)
    r = pl.program_id(0)
    col = jax.lax.broadcasted_iota(jnp.int32, y.shape, 1)
    y = jnp.where((col == DEGCOL) & (r < R), 1.0, y)
    y_ref[0] = y


def _dense1(x, w1all):
    return pl.pallas_call(
        _dense1_body,
        grid=(4, NB),
        in_specs=[
            pl.BlockSpec((BN, D), lambda r, i: (i, 0)),
            pl.BlockSpec((1, D, TW), lambda r, i: (r, 0, 0)),
        ],
        out_specs=pl.BlockSpec((1, BN, TW), lambda r, i: (r, i, 0)),
        out_shape=jax.ShapeDtypeStruct((4, N, TW), jnp.float32),
    )(x, w1all)


def _sc_edge_pass(table, cidx2, dst2, zeros_w):
    mesh = plsc.VectorSubcoreMesh(core_axis_name="c", subcore_axis_name="s")

    @functools.partial(
        pl.kernel,
        mesh=mesh,
        out_type=jax.ShapeDtypeStruct((2, N, TW), jnp.float32),
        scratch_types=[
            pltpu.VMEM((NCH // 2, CHUNK), jnp.int32),
            pltpu.VMEM((NCH // 2, CHUNK), jnp.int32),
            pltpu.VMEM((CHUNK, TW), jnp.float32),
            pltpu.VMEM((CHUNK, TW), jnp.float32),
            pltpu.SemaphoreType.DMA,
            pltpu.SemaphoreType.DMA,
            pltpu.SemaphoreType.DMA,
            pltpu.SemaphoreType.DMA,
            pltpu.VMEM_SHARED((NACC, TW), jnp.float32),
        ],
    )
    def k(table_hbm, cidx_hbm, dst_hbm, zw_hbm, agg_hbm,
          cidx_v, dst_v, rows_v, rows2_v, sem0, sem1, sem2, sem3, acc_sh):
        c = lax.axis_index("c")
        s = lax.axis_index("s")
        w = c * 16 + s

        pltpu.sync_copy(zw_hbm, rows_v)

        @pl.when(s < 15)
        def _():
            @pl.loop(0, 5)
            def _(jz):
                pltpu.sync_copy(rows_v, acc_sh.at[pl.ds(s * 640 + jz * 128,
                                                        128)])

        @pl.when(s == 15)
        def _():
            @pl.loop(0, 3)
            def _(jz):
                pltpu.sync_copy(rows_v, acc_sh.at[pl.ds(9600 + jz * 128,
                                                        128)])
            pltpu.sync_copy(rows_v.at[pl.ds(0, 32)],
                            acc_sh.at[pl.ds(9984, 32)])

        plsc.subcore_barrier()

        HH = NCH // 2
        for h in range(2):
            pltpu.sync_copy(cidx_hbm.at[pl.ds(w * NCH + h * HH, HH)], cidx_v)
            pltpu.sync_copy(dst_hbm.at[pl.ds(w * NCH + h * HH, HH)], dst_v)

            pltpu.make_async_copy(
                table_hbm.at[cidx_v.at[0]], rows_v, sem0).start()
            pltpu.make_async_copy(
                table_hbm.at[cidx_v.at[1]], rows2_v, sem1).start()

            @pl.loop(0, HH, step=2)
            def _(j):
                pltpu.make_async_copy(
                    table_hbm.at[cidx_v.at[0]], rows_v, sem0).wait()
                pltpu.make_async_copy(
                    rows_v, acc_sh.at[dst_v.at[j]], sem2).start(add=True)

                pltpu.make_async_copy(
                    table_hbm.at[cidx_v.at[1]], rows2_v, sem1).wait()
                pltpu.make_async_copy(
                    rows2_v, acc_sh.at[dst_v.at[j + 1]], sem3).start(add=True)

                pltpu.make_async_copy(
                    rows_v, acc_sh.at[dst_v.at[0]], sem2).wait()

                @pl.when(j + 2 < HH)
                def _():
                    pltpu.make_async_copy(
                        table_hbm.at[cidx_v.at[j + 2]], rows_v, sem0).start()

                pltpu.make_async_copy(
                    rows2_v, acc_sh.at[dst_v.at[0]], sem3).wait()

                @pl.when(j + 3 < HH)
                def _():
                    pltpu.make_async_copy(
                        table_hbm.at[cidx_v.at[j + 3]], rows2_v, sem1).start()

        plsc.subcore_barrier()

        @pl.when(s < 15)
        def _():
            @pl.loop(0, 5)
            def _(jz):
                pltpu.sync_copy(acc_sh.at[pl.ds(s * 640 + jz * 128, 128)],
                                rows_v)
                pltpu.sync_copy(rows_v,
                                agg_hbm.at[c, pl.ds(s * 640 + jz * 128, 128)])

        @pl.when(s == 15)
        def _():
            @pl.loop(0, 3)
            def _(jz):
                pltpu.sync_copy(acc_sh.at[pl.ds(9600 + jz * 128, 128)],
                                rows_v)
                pltpu.sync_copy(rows_v,
                                agg_hbm.at[c, pl.ds(9600 + jz * 128, 128)])
            pltpu.sync_copy(acc_sh.at[pl.ds(9984, 16)],
                            rows_v.at[pl.ds(0, 16)])
            pltpu.sync_copy(rows_v.at[pl.ds(0, 16)],
                            agg_hbm.at[c, pl.ds(9984, 16)])

    return k(table, cidx2, dst2, zeros_w)


def _dense2_body(agg_ref, xs_ref, b1_ref, w_ref, y_ref):
    agg = agg_ref[0] + agg_ref[1]
    deg = jnp.maximum(agg[:, DEGCOL:DEGCOL + 1], 1.0)
    h = agg[:, 0:H] / deg + xs_ref[...][:, 0:H] + b1_ref[...]
    h = jnp.maximum(h, 0.0)
    y = jax.lax.dot_general(
        h, w_ref[0], (((1,), (0,)), ((), ())),
        preferred_element_type=jnp.float32)
    r = pl.program_id(0)
    col = jax.lax.broadcasted_iota(jnp.int32, y.shape, 1)
    y = jnp.where((col == DEGCOL) & (r < R), 1.0, y)
    y_ref[0] = y


def _dense2(aggp, xself, b1r, w2all):
    return pl.pallas_call(
        _dense2_body,
        grid=(4, NB),
        in_specs=[
            pl.BlockSpec((2, BN, TW), lambda r, i: (0, i, 0)),
            pl.BlockSpec((BN, TW), lambda r, i: (i, 0)),
            pl.BlockSpec((1, H), lambda r, i: (0, 0)),
            pl.BlockSpec((1, H, TW), lambda r, i: (r, 0, 0)),
        ],
        out_specs=pl.BlockSpec((1, BN, TW), lambda r, i: (r, i, 0)),
        out_shape=jax.ShapeDtypeStruct((4, N, TW), jnp.float32),
    )(aggp, xself, b1r, w2all)


def _final_body(agg2_ref, hs_ref, b2_ref, x_ref, qs_ref, nt_ref,
                gid_ref, wb1_ref, bb1_ref, wb2_ref, bb2_ref,
                ws_ref, q_ref, feat_ref,
                featacc, qacc, snfacc):
    i = pl.program_id(0)

    @pl.when(i == 0)
    def _():
        featacc[...] = jnp.zeros_like(featacc)
        qacc[...] = jnp.zeros_like(qacc)
        snfacc[...] = jnp.zeros_like(snfacc)

    agg2 = agg2_ref[0] + agg2_ref[1]
    deg = jnp.maximum(agg2[:, DEGCOL:DEGCOL + 1], 1.0)
    ws = agg2[:, 0:C] / deg + hs_ref[...][:, 0:C] + b2_ref[...]
    ws_ref[...] = ws

    mask = (nt_ref[...] == 0).astype(jnp.float32)
    bmat = ws * mask
    gid = gid_ref[...]
    iota = jax.lax.broadcasted_iota(jnp.int32, (1, G), 1)
    smt = (gid == iota).astype(jnp.float32)
    x = x_ref[...]

    snfacc[...] += jax.lax.dot_general(
        smt, x, (((0,), (0,)), ((), ())),
        preferred_element_type=jnp.float32)
    qsb = qs_ref[...] * bmat
    qacc[...] += jax.lax.dot_general(
        smt, qsb, (((0,), (0,)), ((), ())),
        preferred_element_type=jnp.float32)
    for cc in range(C):
        v = smt * bmat[:, cc:cc + 1]
        featacc[cc * G:(cc + 1) * G, :] += jax.lax.dot_general(
            v, x, (((0,), (0,)), ((), ())),
            preferred_element_type=jnp.float32)

    @pl.when(i == NB - 1)
    def _():
        z = jax.lax.dot_general(
            snfacc[...], wb1_ref[...], (((1,), (0,)), ((), ())),
            preferred_element_type=jnp.float32) + bb1_ref[...]
        z = jnp.maximum(z, 0.0)
        qv = jax.lax.dot_general(
            z, wb2_ref[...], (((1,), (0,)), ((), ())),
            preferred_element_type=jnp.float32) + bb2_ref[...]
        q_ref[...] = qacc[...] + qv
        feat_ref[...] = featacc[...]


def _final(agg2p, hself2, b2r, x, qs_r, nt_r, gid_r, wb1, bb1r, wb2, bb2r):
    return pl.pallas_call(
        _final_body,
        grid=(NB,),
        in_specs=[
            pl.BlockSpec((2, BN, TW), lambda i: (0, i, 0)),
            pl.BlockSpec((BN, TW), lambda i: (i, 0)),
            pl.BlockSpec((1, C), lambda i: (0, 0)),
            pl.BlockSpec((BN, D), lambda i: (i, 0)),
            pl.BlockSpec((BN, 1), lambda i: (i, 0)),
            pl.BlockSpec((BN, 1), lambda i: (i, 0)),
            pl.BlockSpec((BN, 1), lambda i: (i, 0)),
            pl.BlockSpec((D, H), lambda i: (0, 0)),
            pl.BlockSpec((1, H), lambda i: (0, 0)),
            pl.BlockSpec((H, C), lambda i: (0, 0)),
            pl.BlockSpec((1, C), lambda i: (0, 0)),
        ],
        out_specs=[
            pl.BlockSpec((BN, C), lambda i: (i, 0)),
            pl.BlockSpec((G, C), lambda i: (0, 0)),
            pl.BlockSpec((C * G, D), lambda i: (0, 0)),
        ],
        out_shape=[
            jax.ShapeDtypeStruct((N, C), jnp.float32),
            jax.ShapeDtypeStruct((G, C), jnp.float32),
            jax.ShapeDtypeStruct((C * G, D), jnp.float32),
        ],
        scratch_shapes=[
            pltpu.VMEM((C * G, D), jnp.float32),
            pltpu.VMEM((G, C), jnp.float32),
            pltpu.VMEM((G, D), jnp.float32),
        ],
    )(agg2p, hself2, b2r, x, qs_r, nt_r, gid_r, wb1, bb1r, wb2, bb2r)


def kernel(node_feature, qs, edge_index, edge_type, node_type, graph_ids,
           W_rel1, W_self1, b1, W_rel2, W_self2, b2, Wb1, bb1, Wb2, bb2):
    f32 = jnp.float32
    src = edge_index[0]
    dst = edge_index[1]

    cidx = edge_type * N + src
    pad = EPAD - E
    cidx2 = jnp.concatenate([cidx, jnp.zeros((pad,), jnp.int32)])
    cidx2 = cidx2.reshape(NW * NCH, CHUNK)
    dst2 = jnp.concatenate([dst, jnp.full((pad,), N, jnp.int32)])
    dst2 = dst2.reshape(NW * NCH, CHUNK)

    zeros_w = jnp.zeros((CHUNK, TW), f32)

    w1all = jnp.zeros((4, D, TW), f32)
    w1all = w1all.at[:R, :, :H].set(W_rel1)
    w1all = w1all.at[R, :, :H].set(W_self1)
    y1 = _dense1(node_feature, w1all)
    table1 = y1[:R].reshape(R * N, TW)
    xself1 = y1[R]

    agg1p = _sc_edge_pass(table1, cidx2, dst2, zeros_w)

    w2all = jnp.zeros((4, H, TW), f32)
    w2all = w2all.at[:R, :, :C].set(W_rel2)
    w2all = w2all.at[R, :, :C].set(W_self2)
    y2 = _dense2(agg1p, xself1, b1.reshape(1, H), w2all)
    table2 = y2[:R].reshape(R * N, TW)
    hself2 = y2[R]

    agg2p = _sc_edge_pass(table2, cidx2, dst2, zeros_w)

    ws, q_aggregated, featflat = _final(
        agg2p, hself2, b2.reshape(1, C), node_feature,
        qs.reshape(N, 1), node_type.reshape(N, 1), graph_ids.reshape(N, 1),
        Wb1, bb1.reshape(1, H), Wb2, bb2.reshape(1, C))

    feat = featflat.reshape(C, G, D).transpose(1, 0, 2)
    return (q_aggregated, ws, feat)


# speedup vs baseline: 2.5695x; 2.5695x over previous
import functools

import jax
import jax.numpy as jnp
from jax import lax
from jax.experimental import pallas as pl
from jax.experimental.pallas import tpu as pltpu
from jax.experimental.pallas import tpu_sc as plsc

N = 10000
E = 320000
D = 128
H = 64
C = 4
R = 3
G = 32

NW = 32
CHUNK = 128
NCH = 80
EPAD = NW * NCH * CHUNK
NACC = N + 16
TW = 128
DEGCOL = H

BN = 2000
NB = N // BN


def _dense1_body(x_ref, w_ref, y_ref):
    y = jax.lax.dot_general(
        x_ref[...], w_ref[0], (((1,), (0,)), ((), ())),
        preferred_element_type=jnp.float32)
    r = pl.program_id(0)
    col = jax.lax.broadcasted_iota(jnp.int32, y.shape, 1)
    y = jnp.where((col == DEGCOL) & (r < R), 1.0, y)
    y_ref[0] = y


def _dense1(x, w1all):
    return pl.pallas_call(
        _dense1_body,
        grid=(4, NB),
        in_specs=[
            pl.BlockSpec((BN, D), lambda r, i: (i, 0)),
            pl.BlockSpec((1, D, TW), lambda r, i: (r, 0, 0)),
        ],
        out_specs=pl.BlockSpec((1, BN, TW), lambda r, i: (r, i, 0)),
        out_shape=jax.ShapeDtypeStruct((4, N, TW), jnp.float32),
    )(x, w1all)


def _sc_edge_pass(table, cidx2, dst2, zeros_w):
    mesh = plsc.VectorSubcoreMesh(core_axis_name="c", subcore_axis_name="s")

    @functools.partial(
        pl.kernel,
        mesh=mesh,
        out_type=jax.ShapeDtypeStruct((2, N, TW), jnp.float32),
        scratch_types=[
            pltpu.VMEM((NCH // 2, CHUNK), jnp.int32),
            pltpu.VMEM((NCH // 2, CHUNK), jnp.int32),
            pltpu.VMEM((CHUNK, TW), jnp.float32),
            pltpu.VMEM((CHUNK, TW), jnp.float32),
            pltpu.SemaphoreType.DMA,
            pltpu.SemaphoreType.DMA,
            pltpu.SemaphoreType.DMA,
            pltpu.SemaphoreType.DMA,
            pltpu.VMEM_SHARED((NACC, TW), jnp.float32),
        ],
    )
    def k(table_hbm, cidx_hbm, dst_hbm, zw_hbm, agg_hbm,
          cidx_v, dst_v, rows_v, rows2_v, sem0, sem1, sem2, sem3, acc_sh):
        c = lax.axis_index("c")
        s = lax.axis_index("s")
        w = c * 16 + s

        pltpu.sync_copy(zw_hbm, rows_v)

        @pl.when(s < 15)
        def _():
            @pl.loop(0, 5)
            def _(jz):
                pltpu.sync_copy(rows_v, acc_sh.at[pl.ds(s * 640 + jz * 128,
                                                        128)])

        @pl.when(s == 15)
        def _():
            @pl.loop(0, 3)
            def _(jz):
                pltpu.sync_copy(rows_v, acc_sh.at[pl.ds(9600 + jz * 128,
                                                        128)])
            pltpu.sync_copy(rows_v.at[pl.ds(0, 32)],
                            acc_sh.at[pl.ds(9984, 32)])

        plsc.subcore_barrier()

        HH = NCH // 2
        for h in range(2):
            pltpu.sync_copy(cidx_hbm.at[pl.ds(w * NCH + h * HH, HH)], cidx_v)
            pltpu.sync_copy(dst_hbm.at[pl.ds(w * NCH + h * HH, HH)], dst_v)

            pltpu.make_async_copy(
                table_hbm.at[cidx_v.at[0]], rows_v, sem0).start()
            pltpu.make_async_copy(
                table_hbm.at[cidx_v.at[1]], rows2_v, sem1).start()

            @pl.loop(0, HH, step=2)
            def _(j):
                pltpu.make_async_copy(
                    table_hbm.at[cidx_v.at[0]], rows_v, sem0).wait()
                pltpu.make_async_copy(
                    rows_v, acc_sh.at[dst_v.at[j]], sem2).start(add=True)

                pltpu.make_async_copy(
                    table_hbm.at[cidx_v.at[1]], rows2_v, sem1).wait()
                pltpu.make_async_copy(
                    rows2_v, acc_sh.at[dst_v.at[j + 1]], sem3).start(add=True)

                pltpu.make_async_copy(
                    rows_v, acc_sh.at[dst_v.at[0]], sem2).wait()

                @pl.when(j + 2 < HH)
                def _():
                    pltpu.make_async_copy(
                        table_hbm.at[cidx_v.at[j + 2]], rows_v, sem0).start()

                pltpu.make_async_copy(
                    rows2_v, acc_sh.at[dst_v.at[0]], sem3).wait()

                @pl.when(j + 3 < HH)
                def _():
                    pltpu.make_async_copy(
                        table_hbm.at[cidx_v.at[j + 3]], rows2_v, sem1).start()

        plsc.subcore_barrier()

        @pl.when(s < 15)
        def _():
            @pl.loop(0, 5)
            def _(jz):
                pltpu.sync_copy(acc_sh.at[pl.ds(s * 640 + jz * 128, 128)],
                                rows_v)
                pltpu.sync_copy(rows_v,
                                agg_hbm.at[c, pl.ds(s * 640 + jz * 128, 128)])

        @pl.when(s == 15)
        def _():
            @pl.loop(0, 3)
            def _(jz):
                pltpu.sync_copy(acc_sh.at[pl.ds(9600 + jz * 128, 128)],
                                rows_v)
                pltpu.sync_copy(rows_v,
                                agg_hbm.at[c, pl.ds(9600 + jz * 128, 128)])
            pltpu.sync_copy(acc_sh.at[pl.ds(9984, 16)],
                            rows_v.at[pl.ds(0, 16)])
            pltpu.sync_copy(rows_v.at[pl.ds(0, 16)],
                            agg_hbm.at[c, pl.ds(9984, 16)])

    return k(table, cidx2, dst2, zeros_w)


def _dense2_body(agg_ref, xs_ref, b1_ref, w_ref, y_ref):
    agg = agg_ref[0] + agg_ref[1]
    deg = jnp.maximum(agg[:, DEGCOL:DEGCOL + 1], 1.0)
    h = agg[:, 0:H] / deg + xs_ref[...][:, 0:H] + b1_ref[...]
    h = jnp.maximum(h, 0.0)
    y = jax.lax.dot_general(
        h, w_ref[0], (((1,), (0,)), ((), ())),
        preferred_element_type=jnp.float32)
    r = pl.program_id(0)
    col = jax.lax.broadcasted_iota(jnp.int32, y.shape, 1)
    y = jnp.where((col == DEGCOL) & (r < R), 1.0, y)
    y_ref[0] = y


def _dense2(aggp, xself, b1r, w2all):
    return pl.pallas_call(
        _dense2_body,
        grid=(4, NB),
        in_specs=[
            pl.BlockSpec((2, BN, TW), lambda r, i: (0, i, 0)),
            pl.BlockSpec((BN, TW), lambda r, i: (i, 0)),
            pl.BlockSpec((1, H), lambda r, i: (0, 0)),
            pl.BlockSpec((1, H, TW), lambda r, i: (r, 0, 0)),
        ],
        out_specs=pl.BlockSpec((1, BN, TW), lambda r, i: (r, i, 0)),
        out_shape=jax.ShapeDtypeStruct((4, N, TW), jnp.float32),
    )(aggp, xself, b1r, w2all)


def _final_body(agg2_ref, hs_ref, b2_ref, x_ref, qs_ref, nt_ref,
                gid_ref, wb1_ref, bb1_ref, wb2_ref, bb2_ref,
                ws_ref, q_ref, feat_ref,
                featacc, qacc, snfacc):
    i = pl.program_id(0)

    @pl.when(i == 0)
    def _():
        featacc[...] = jnp.zeros_like(featacc)
        qacc[...] = jnp.zeros_like(qacc)
        snfacc[...] = jnp.zeros_like(snfacc)

    agg2 = agg2_ref[0] + agg2_ref[1]
    deg = jnp.maximum(agg2[:, DEGCOL:DEGCOL + 1], 1.0)
    ws = agg2[:, 0:C] / deg + hs_ref[...][:, 0:C] + b2_ref[...]
    ws_ref[...] = ws

    mask = (nt_ref[...] == 0).astype(jnp.float32)
    bmat = ws * mask
    gid = gid_ref[...]
    iota = jax.lax.broadcasted_iota(jnp.int32, (1, G), 1)
    smt = (gid == iota).astype(jnp.float32)
    x = x_ref[...]

    snfacc[...] += jax.lax.dot_general(
        smt, x, (((0,), (0,)), ((), ())),
        preferred_element_type=jnp.float32)
    qsb = qs_ref[...] * bmat
    qacc[...] += jax.lax.dot_general(
        smt, qsb, (((0,), (0,)), ((), ())),
        preferred_element_type=jnp.float32)
    for cc in range(C):
        v = smt * bmat[:, cc:cc + 1]
        featacc[cc * G:(cc + 1) * G, :] += jax.lax.dot_general(
            v, x, (((0,), (0,)), ((), ())),
            preferred_element_type=jnp.float32)

    @pl.when(i == NB - 1)
    def _():
        z = jax.lax.dot_general(
            snfacc[...], wb1_ref[...], (((1,), (0,)), ((), ())),
            preferred_element_type=jnp.float32) + bb1_ref[...]
        z = jnp.maximum(z, 0.0)
        qv = jax.lax.dot_general(
            z, wb2_ref[...], (((1,), (0,)), ((), ())),
            preferred_element_type=jnp.float32) + bb2_ref[...]
        q_ref[...] = qacc[...] + qv
        feat_ref[...] = featacc[...]


def _final(agg2p, hself2, b2r, x, qs_r, nt_r, gid_r, wb1, bb1r, wb2, bb2r):
    return pl.pallas_call(
        _final_body,
        grid=(NB,),
        in_specs=[
            pl.BlockSpec((2, BN, TW), lambda i: (0, i, 0)),
            pl.BlockSpec((BN, TW), lambda i: (i, 0)),
            pl.BlockSpec((1, C), lambda i: (0, 0)),
            pl.BlockSpec((BN, D), lambda i: (i, 0)),
            pl.BlockSpec((BN, 1), lambda i: (i, 0)),
            pl.BlockSpec((BN, 1), lambda i: (i, 0)),
            pl.BlockSpec((BN, 1), lambda i: (i, 0)),
            pl.BlockSpec((D, H), lambda i: (0, 0)),
            pl.BlockSpec((1, H), lambda i: (0, 0)),
            pl.BlockSpec((H, C), lambda i: (0, 0)),
            pl.BlockSpec((1, C), lambda i: (0, 0)),
        ],
        out_specs=[
            pl.BlockSpec((BN, C), lambda i: (i, 0)),
            pl.BlockSpec((G, C), lambda i: (0, 0)),
            pl.BlockSpec((C * G, D), lambda i: (0, 0)),
        ],
        out_shape=[
            jax.ShapeDtypeStruct((N, C), jnp.float32),
            jax.ShapeDtypeStruct((G, C), jnp.float32),
            jax.ShapeDtypeStruct((C * G, D), jnp.float32),
        ],
        scratch_shapes=[
            pltpu.VMEM((C * G, D), jnp.float32),
            pltpu.VMEM((G, C), jnp.float32),
            pltpu.VMEM((G, D), jnp.float32),
        ],
    )(agg2p, hself2, b2r, x, qs_r, nt_r, gid_r, wb1, bb1r, wb2, bb2r)


def kernel(node_feature, qs, edge_index, edge_type, node_type, graph_ids,
           W_rel1, W_self1, b1, W_rel2, W_self2, b2, Wb1, bb1, Wb2, bb2):
    f32 = jnp.float32
    src = edge_index[0]
    dst = edge_index[1]

    cidx = edge_type * N + src
    pad = EPAD - E
    ar = jnp.arange(pad, dtype=jnp.int32)
    cidx2 = jnp.concatenate([cidx, ar % (R * N)])
    cidx2 = cidx2.reshape(NW * NCH, CHUNK)
    dst2 = jnp.concatenate([dst, N + (ar % 16)])
    dst2 = dst2.reshape(NW * NCH, CHUNK)

    zeros_w = jnp.zeros((CHUNK, TW), f32)

    w1all = jnp.zeros((4, D, TW), f32)
    w1all = w1all.at[:R, :, :H].set(W_rel1)
    w1all = w1all.at[R, :, :H].set(W_self1)
    y1 = _dense1(node_feature, w1all)
    table1 = y1[:R].reshape(R * N, TW)
    xself1 = y1[R]

    agg1p = _sc_edge_pass(table1, cidx2, dst2, zeros_w)

    w2all = jnp.zeros((4, H, TW), f32)
    w2all = w2all.at[:R, :, :C].set(W_rel2)
    w2all = w2all.at[R, :, :C].set(W_self2)
    y2 = _dense2(agg1p, xself1, b1.reshape(1, H), w2all)
    table2 = y2[:R].reshape(R * N, TW)
    hself2 = y2[R]

    agg2p = _sc_edge_pass(table2, cidx2, dst2, zeros_w)

    ws, q_aggregated, featflat = _final(
        agg2p, hself2, b2.reshape(1, C), node_feature,
        qs.reshape(N, 1), node_type.reshape(N, 1), graph_ids.reshape(N, 1),
        Wb1, bb1.reshape(1, H), Wb2, bb2.reshape(1, C))

    feat = featflat.reshape(C, G, D).transpose(1, 0, 2)
    return (q_aggregated, ws, feat)


# speedup vs baseline: 2.7351x; 1.0644x over previous
import functools

import jax
import jax.numpy as jnp
from jax import lax
from jax.experimental import pallas as pl
from jax.experimental.pallas import tpu as pltpu
from jax.experimental.pallas import tpu_sc as plsc

N = 10000
E = 320000
D = 128
H = 64
C = 4
R = 3
G = 32

NW = 32
CHUNK = 128
NCH = 80
EPAD = NW * NCH * CHUNK
NACC = N + 16
TW = 128
DEGCOL = H

BN = 2000
NB = N // BN


def _dense1_body(x_ref, w_ref, y_ref):
    y = jax.lax.dot_general(
        x_ref[...], w_ref[0], (((1,), (0,)), ((), ())),
        preferred_element_type=jnp.float32)
    r = pl.program_id(0)
    col = jax.lax.broadcasted_iota(jnp.int32, y.shape, 1)
    y = jnp.where((col == DEGCOL) & (r < R), 1.0, y)
    y_ref[...] = y


def _dense1(x, w1all):
    return pl.pallas_call(
        _dense1_body,
        grid=(4, NB),
        in_specs=[
            pl.BlockSpec((BN, D), lambda r, i: (i, 0)),
            pl.BlockSpec((1, D, TW), lambda r, i: (r, 0, 0)),
        ],
        out_specs=pl.BlockSpec((BN, TW), lambda r, i: (r * NB + i, 0)),
        out_shape=jax.ShapeDtypeStruct((4 * N, TW), jnp.float32),
    )(x, w1all)


def _sc_edge_pass(table, cidx2, dst2, zeros_w):
    mesh = plsc.VectorSubcoreMesh(core_axis_name="c", subcore_axis_name="s")

    @functools.partial(
        pl.kernel,
        mesh=mesh,
        out_type=jax.ShapeDtypeStruct((2, N, TW), jnp.float32),
        scratch_types=[
            pltpu.VMEM((NCH // 2, CHUNK), jnp.int32),
            pltpu.VMEM((NCH // 2, CHUNK), jnp.int32),
            pltpu.VMEM((CHUNK, TW), jnp.float32),
            pltpu.VMEM((CHUNK, TW), jnp.float32),
            pltpu.SemaphoreType.DMA,
            pltpu.SemaphoreType.DMA,
            pltpu.SemaphoreType.DMA,
            pltpu.SemaphoreType.DMA,
            pltpu.VMEM_SHARED((NACC, TW), jnp.float32),
        ],
    )
    def k(table_hbm, cidx_hbm, dst_hbm, zw_hbm, agg_hbm,
          cidx_v, dst_v, rows_v, rows2_v, sem0, sem1, sem2, sem3, acc_sh):
        c = lax.axis_index("c")
        s = lax.axis_index("s")
        w = c * 16 + s

        pltpu.sync_copy(zw_hbm, rows_v)

        @pl.when(s < 15)
        def _():
            @pl.loop(0, 5)
            def _(jz):
                pltpu.sync_copy(rows_v, acc_sh.at[pl.ds(s * 640 + jz * 128,
                                                        128)])

        @pl.when(s == 15)
        def _():
            @pl.loop(0, 3)
            def _(jz):
                pltpu.sync_copy(rows_v, acc_sh.at[pl.ds(9600 + jz * 128,
                                                        128)])
            pltpu.sync_copy(rows_v.at[pl.ds(0, 32)],
                            acc_sh.at[pl.ds(9984, 32)])

        plsc.subcore_barrier()

        HH = NCH // 2
        for h in range(2):
            pltpu.sync_copy(cidx_hbm.at[pl.ds(w * NCH + h * HH, HH)], cidx_v)
            pltpu.sync_copy(dst_hbm.at[pl.ds(w * NCH + h * HH, HH)], dst_v)

            pltpu.make_async_copy(
                table_hbm.at[cidx_v.at[0]], rows_v, sem0).start()
            pltpu.make_async_copy(
                table_hbm.at[cidx_v.at[1]], rows2_v, sem1).start()

            @pl.loop(0, HH, step=2)
            def _(j):
                pltpu.make_async_copy(
                    table_hbm.at[cidx_v.at[0]], rows_v, sem0).wait()
                pltpu.make_async_copy(
                    rows_v, acc_sh.at[dst_v.at[j]], sem2).start(add=True)

                pltpu.make_async_copy(
                    table_hbm.at[cidx_v.at[1]], rows2_v, sem1).wait()
                pltpu.make_async_copy(
                    rows2_v, acc_sh.at[dst_v.at[j + 1]], sem3).start(add=True)

                pltpu.make_async_copy(
                    rows_v, acc_sh.at[dst_v.at[0]], sem2).wait()

                @pl.when(j + 2 < HH)
                def _():
                    pltpu.make_async_copy(
                        table_hbm.at[cidx_v.at[j + 2]], rows_v, sem0).start()

                pltpu.make_async_copy(
                    rows2_v, acc_sh.at[dst_v.at[0]], sem3).wait()

                @pl.when(j + 3 < HH)
                def _():
                    pltpu.make_async_copy(
                        table_hbm.at[cidx_v.at[j + 3]], rows2_v, sem1).start()

        plsc.subcore_barrier()

        @pl.when(s < 15)
        def _():
            @pl.loop(0, 5)
            def _(jz):
                pltpu.sync_copy(acc_sh.at[pl.ds(s * 640 + jz * 128, 128)],
                                rows_v)
                pltpu.sync_copy(rows_v,
                                agg_hbm.at[c, pl.ds(s * 640 + jz * 128, 128)])

        @pl.when(s == 15)
        def _():
            @pl.loop(0, 3)
            def _(jz):
                pltpu.sync_copy(acc_sh.at[pl.ds(9600 + jz * 128, 128)],
                                rows_v)
                pltpu.sync_copy(rows_v,
                                agg_hbm.at[c, pl.ds(9600 + jz * 128, 128)])
            pltpu.sync_copy(acc_sh.at[pl.ds(9984, 16)],
                            rows_v.at[pl.ds(0, 16)])
            pltpu.sync_copy(rows_v.at[pl.ds(0, 16)],
                            agg_hbm.at[c, pl.ds(9984, 16)])

    return k(table, cidx2, dst2, zeros_w)


def _dense2_body(agg_ref, xs_ref, b1_ref, w_ref, y_ref):
    agg = agg_ref[0] + agg_ref[1]
    deg = jnp.maximum(agg[:, DEGCOL:DEGCOL + 1], 1.0)
    h = agg[:, 0:H] / deg + xs_ref[...][:, 0:H] + b1_ref[...]
    h = jnp.maximum(h, 0.0)
    y = jax.lax.dot_general(
        h, w_ref[0], (((1,), (0,)), ((), ())),
        preferred_element_type=jnp.float32)
    r = pl.program_id(0)
    col = jax.lax.broadcasted_iota(jnp.int32, y.shape, 1)
    y = jnp.where((col == DEGCOL) & (r < R), 1.0, y)
    y_ref[...] = y


def _dense2(aggp, xself, b1r, w2all):
    return pl.pallas_call(
        _dense2_body,
        grid=(4, NB),
        in_specs=[
            pl.BlockSpec((2, BN, TW), lambda r, i: (0, i, 0)),
            pl.BlockSpec((BN, TW), lambda r, i: (R * NB + i, 0)),
            pl.BlockSpec((1, H), lambda r, i: (0, 0)),
            pl.BlockSpec((1, H, TW), lambda r, i: (r, 0, 0)),
        ],
        out_specs=pl.BlockSpec((BN, TW), lambda r, i: (r * NB + i, 0)),
        out_shape=jax.ShapeDtypeStruct((4 * N, TW), jnp.float32),
    )(aggp, xself, b1r, w2all)


def _final_body(agg2_ref, hs_ref, b2_ref, x_ref, qs_ref, nt_ref,
                gid_ref, wb1_ref, bb1_ref, wb2_ref, bb2_ref,
                ws_ref, q_ref, feat_ref,
                featacc, qacc, snfacc):
    i = pl.program_id(0)

    @pl.when(i == 0)
    def _():
        featacc[...] = jnp.zeros_like(featacc)
        qacc[...] = jnp.zeros_like(qacc)
        snfacc[...] = jnp.zeros_like(snfacc)

    agg2 = agg2_ref[0] + agg2_ref[1]
    deg = jnp.maximum(agg2[:, DEGCOL:DEGCOL + 1], 1.0)
    ws = agg2[:, 0:C] / deg + hs_ref[...][:, 0:C] + b2_ref[...]
    ws_ref[...] = ws

    mask = (nt_ref[...] == 0).astype(jnp.float32)
    bmat = ws * mask
    gid = gid_ref[...]
    iota = jax.lax.broadcasted_iota(jnp.int32, (1, G), 1)
    smt = (gid == iota).astype(jnp.float32)
    x = x_ref[...]

    snfacc[...] += jax.lax.dot_general(
        smt, x, (((0,), (0,)), ((), ())),
        preferred_element_type=jnp.float32)
    qsb = qs_ref[...] * bmat
    qacc[...] += jax.lax.dot_general(
        smt, qsb, (((0,), (0,)), ((), ())),
        preferred_element_type=jnp.float32)
    for cc in range(C):
        v = smt * bmat[:, cc:cc + 1]
        featacc[cc * G:(cc + 1) * G, :] += jax.lax.dot_general(
            v, x, (((0,), (0,)), ((), ())),
            preferred_element_type=jnp.float32)

    @pl.when(i == NB - 1)
    def _():
        z = jax.lax.dot_general(
            snfacc[...], wb1_ref[...], (((1,), (0,)), ((), ())),
            preferred_element_type=jnp.float32) + bb1_ref[...]
        z = jnp.maximum(z, 0.0)
        qv = jax.lax.dot_general(
            z, wb2_ref[...], (((1,), (0,)), ((), ())),
            preferred_element_type=jnp.float32) + bb2_ref[...]
        q_ref[...] = qacc[...] + qv
        feat_ref[...] = featacc[...]


def _final(agg2p, hself2, b2r, x, qs_r, nt_r, gid_r, wb1, bb1r, wb2, bb2r):
    return pl.pallas_call(
        _final_body,
        grid=(NB,),
        in_specs=[
            pl.BlockSpec((2, BN, TW), lambda i: (0, i, 0)),
            pl.BlockSpec((BN, TW), lambda i: (R * NB + i, 0)),
            pl.BlockSpec((1, C), lambda i: (0, 0)),
            pl.BlockSpec((BN, D), lambda i: (i, 0)),
            pl.BlockSpec((BN, 1), lambda i: (i, 0)),
            pl.BlockSpec((BN, 1), lambda i: (i, 0)),
            pl.BlockSpec((BN, 1), lambda i: (i, 0)),
            pl.BlockSpec((D, H), lambda i: (0, 0)),
            pl.BlockSpec((1, H), lambda i: (0, 0)),
            pl.BlockSpec((H, C), lambda i: (0, 0)),
            pl.BlockSpec((1, C), lambda i: (0, 0)),
        ],
        out_specs=[
            pl.BlockSpec((BN, C), lambda i: (i, 0)),
            pl.BlockSpec((G, C), lambda i: (0, 0)),
            pl.BlockSpec((C * G, D), lambda i: (0, 0)),
        ],
        out_shape=[
            jax.ShapeDtypeStruct((N, C), jnp.float32),
            jax.ShapeDtypeStruct((G, C), jnp.float32),
            jax.ShapeDtypeStruct((C * G, D), jnp.float32),
        ],
        scratch_shapes=[
            pltpu.VMEM((C * G, D), jnp.float32),
            pltpu.VMEM((G, C), jnp.float32),
            pltpu.VMEM((G, D), jnp.float32),
        ],
    )(agg2p, hself2, b2r, x, qs_r, nt_r, gid_r, wb1, bb1r, wb2, bb2r)


def kernel(node_feature, qs, edge_index, edge_type, node_type, graph_ids,
           W_rel1, W_self1, b1, W_rel2, W_self2, b2, Wb1, bb1, Wb2, bb2):
    f32 = jnp.float32
    src = edge_index[0]
    dst = edge_index[1]

    cidx = edge_type * N + src
    pad = EPAD - E
    ar = jnp.arange(pad, dtype=jnp.int32)
    cidx2 = jnp.concatenate([cidx, ar % (R * N)])
    cidx2 = cidx2.reshape(NW * NCH, CHUNK)
    dst2 = jnp.concatenate([dst, N + (ar % 16)])
    dst2 = dst2.reshape(NW * NCH, CHUNK)

    zeros_w = jnp.zeros((CHUNK, TW), f32)

    w1all = jnp.zeros((4, D, TW), f32)
    w1all = w1all.at[:R, :, :H].set(W_rel1)
    w1all = w1all.at[R, :, :H].set(W_self1)
    table1p = _dense1(node_feature, w1all)

    agg1p = _sc_edge_pass(table1p, cidx2, dst2, zeros_w)

    w2all = jnp.zeros((4, H, TW), f32)
    w2all = w2all.at[:R, :, :C].set(W_rel2)
    w2all = w2all.at[R, :, :C].set(W_self2)
    table2p = _dense2(agg1p, table1p, b1.reshape(1, H), w2all)

    agg2p = _sc_edge_pass(table2p, cidx2, dst2, zeros_w)

    ws, q_aggregated, featflat = _final(
        agg2p, table2p, b2.reshape(1, C), node_feature,
        qs.reshape(N, 1), node_type.reshape(N, 1), graph_ids.reshape(N, 1),
        Wb1, bb1.reshape(1, H), Wb2, bb2.reshape(1, C))

    feat = featflat.reshape(C, G, D).transpose(1, 0, 2)
    return (q_aggregated, ws, feat)


# speedup vs baseline: 2.9871x; 1.0922x over previous
import functools

import jax
import jax.numpy as jnp
from jax import lax
from jax.experimental import pallas as pl
from jax.experimental.pallas import tpu as pltpu
from jax.experimental.pallas import tpu_sc as plsc

N = 10000
E = 320000
D = 128
H = 64
C = 4
R = 3
G = 32

NW = 32
CHUNK = 128
NCH = 80
EPAD = NW * NCH * CHUNK
NACC = N + 16
TW = 72
DEGCOL = H

BN = 2000
NB = N // BN


def _dense1_body(x_ref, w_ref, y_ref):
    y = jax.lax.dot_general(
        x_ref[...], w_ref[0], (((1,), (0,)), ((), ())),
        preferred_element_type=jnp.float32)
    r = pl.program_id(0)
    col = jax.lax.broadcasted_iota(jnp.int32, y.shape, 1)
    y = jnp.where((col == DEGCOL) & (r < R), 1.0, y)
    y_ref[...] = y


def _dense1(x, w1all):
    return pl.pallas_call(
        _dense1_body,
        grid=(4, NB),
        in_specs=[
            pl.BlockSpec((BN, D), lambda r, i: (i, 0)),
            pl.BlockSpec((1, D, TW), lambda r, i: (r, 0, 0)),
        ],
        out_specs=pl.BlockSpec((BN, TW), lambda r, i: (r * NB + i, 0)),
        out_shape=jax.ShapeDtypeStruct((4 * N, TW), jnp.float32),
    )(x, w1all)


def _sc_edge_pass(table, cidx2, dst2, zeros_w):
    mesh = plsc.VectorSubcoreMesh(core_axis_name="c", subcore_axis_name="s")

    @functools.partial(
        pl.kernel,
        mesh=mesh,
        compiler_params=pltpu.CompilerParams(use_tc_tiling_on_sc=False),
        out_type=jax.ShapeDtypeStruct((2, N, TW), jnp.float32),
        scratch_types=[
            pltpu.VMEM((NCH // 2, CHUNK), jnp.int32),
            pltpu.VMEM((NCH // 2, CHUNK), jnp.int32),
            pltpu.VMEM((CHUNK, TW), jnp.float32),
            pltpu.VMEM((CHUNK, TW), jnp.float32),
            pltpu.SemaphoreType.DMA,
            pltpu.SemaphoreType.DMA,
            pltpu.SemaphoreType.DMA,
            pltpu.SemaphoreType.DMA,
            pltpu.VMEM_SHARED((NACC, TW), jnp.float32),
        ],
    )
    def k(table_hbm, cidx_hbm, dst_hbm, zw_hbm, agg_hbm,
          cidx_v, dst_v, rows_v, rows2_v, sem0, sem1, sem2, sem3, acc_sh):
        c = lax.axis_index("c")
        s = lax.axis_index("s")
        w = c * 16 + s

        pltpu.sync_copy(zw_hbm, rows_v)

        @pl.when(s < 15)
        def _():
            @pl.loop(0, 5)
            def _(jz):
                pltpu.sync_copy(rows_v, acc_sh.at[pl.ds(s * 640 + jz * 128,
                                                        128)])

        @pl.when(s == 15)
        def _():
            @pl.loop(0, 3)
            def _(jz):
                pltpu.sync_copy(rows_v, acc_sh.at[pl.ds(9600 + jz * 128,
                                                        128)])
            pltpu.sync_copy(rows_v.at[pl.ds(0, 32)],
                            acc_sh.at[pl.ds(9984, 32)])

        plsc.subcore_barrier()

        HH = NCH // 2
        for h in range(2):
            pltpu.sync_copy(cidx_hbm.at[pl.ds(w * NCH + h * HH, HH)], cidx_v)
            pltpu.sync_copy(dst_hbm.at[pl.ds(w * NCH + h * HH, HH)], dst_v)

            pltpu.make_async_copy(
                table_hbm.at[cidx_v.at[0]], rows_v, sem0).start()
            pltpu.make_async_copy(
                table_hbm.at[cidx_v.at[1]], rows2_v, sem1).start()

            @pl.loop(0, HH, step=2)
            def _(j):
                pltpu.make_async_copy(
                    table_hbm.at[cidx_v.at[0]], rows_v, sem0).wait()
                pltpu.make_async_copy(
                    rows_v, acc_sh.at[dst_v.at[j]], sem2).start(add=True)

                pltpu.make_async_copy(
                    table_hbm.at[cidx_v.at[1]], rows2_v, sem1).wait()
                pltpu.make_async_copy(
                    rows2_v, acc_sh.at[dst_v.at[j + 1]], sem3).start(add=True)

                pltpu.make_async_copy(
                    rows_v, acc_sh.at[dst_v.at[0]], sem2).wait()

                @pl.when(j + 2 < HH)
                def _():
                    pltpu.make_async_copy(
                        table_hbm.at[cidx_v.at[j + 2]], rows_v, sem0).start()

                pltpu.make_async_copy(
                    rows2_v, acc_sh.at[dst_v.at[0]], sem3).wait()

                @pl.when(j + 3 < HH)
                def _():
                    pltpu.make_async_copy(
                        table_hbm.at[cidx_v.at[j + 3]], rows2_v, sem1).start()

        plsc.subcore_barrier()

        @pl.when(s < 15)
        def _():
            @pl.loop(0, 5)
            def _(jz):
                pltpu.sync_copy(acc_sh.at[pl.ds(s * 640 + jz * 128, 128)],
                                rows_v)
                pltpu.sync_copy(rows_v,
                                agg_hbm.at[c, pl.ds(s * 640 + jz * 128, 128)])

        @pl.when(s == 15)
        def _():
            @pl.loop(0, 3)
            def _(jz):
                pltpu.sync_copy(acc_sh.at[pl.ds(9600 + jz * 128, 128)],
                                rows_v)
                pltpu.sync_copy(rows_v,
                                agg_hbm.at[c, pl.ds(9600 + jz * 128, 128)])
            pltpu.sync_copy(acc_sh.at[pl.ds(9984, 16)],
                            rows_v.at[pl.ds(0, 16)])
            pltpu.sync_copy(rows_v.at[pl.ds(0, 16)],
                            agg_hbm.at[c, pl.ds(9984, 16)])

    return k(table, cidx2, dst2, zeros_w)


def _dense2_body(agg_ref, xs_ref, b1_ref, w_ref, y_ref):
    agg = agg_ref[0] + agg_ref[1]
    deg = jnp.maximum(agg[:, DEGCOL:DEGCOL + 1], 1.0)
    h = agg[:, 0:H] / deg + xs_ref[...][:, 0:H] + b1_ref[...]
    h = jnp.maximum(h, 0.0)
    y = jax.lax.dot_general(
        h, w_ref[0], (((1,), (0,)), ((), ())),
        preferred_element_type=jnp.float32)
    r = pl.program_id(0)
    col = jax.lax.broadcasted_iota(jnp.int32, y.shape, 1)
    y = jnp.where((col == DEGCOL) & (r < R), 1.0, y)
    y_ref[...] = y


def _dense2(aggp, xself, b1r, w2all):
    return pl.pallas_call(
        _dense2_body,
        grid=(4, NB),
        in_specs=[
            pl.BlockSpec((2, BN, TW), lambda r, i: (0, i, 0)),
            pl.BlockSpec((BN, TW), lambda r, i: (R * NB + i, 0)),
            pl.BlockSpec((1, H), lambda r, i: (0, 0)),
            pl.BlockSpec((1, H, TW), lambda r, i: (r, 0, 0)),
        ],
        out_specs=pl.BlockSpec((BN, TW), lambda r, i: (r * NB + i, 0)),
        out_shape=jax.ShapeDtypeStruct((4 * N, TW), jnp.float32),
    )(aggp, xself, b1r, w2all)


def _final_body(agg2_ref, hs_ref, b2_ref, x_ref, qs_ref, nt_ref,
                gid_ref, wb1_ref, bb1_ref, wb2_ref, bb2_ref,
                ws_ref, q_ref, feat_ref,
                featacc, qacc, snfacc):
    i = pl.program_id(0)

    @pl.when(i == 0)
    def _():
        featacc[...] = jnp.zeros_like(featacc)
        qacc[...] = jnp.zeros_like(qacc)
        snfacc[...] = jnp.zeros_like(snfacc)

    agg2 = agg2_ref[0] + agg2_ref[1]
    deg = jnp.maximum(agg2[:, DEGCOL:DEGCOL + 1], 1.0)
    ws = agg2[:, 0:C] / deg + hs_ref[...][:, 0:C] + b2_ref[...]
    ws_ref[...] = ws

    mask = (nt_ref[...] == 0).astype(jnp.float32)
    bmat = ws * mask
    gid = gid_ref[...]
    iota = jax.lax.broadcasted_iota(jnp.int32, (1, G), 1)
    smt = (gid == iota).astype(jnp.float32)
    x = x_ref[...]

    snfacc[...] += jax.lax.dot_general(
        smt, x, (((0,), (0,)), ((), ())),
        preferred_element_type=jnp.float32)
    qsb = qs_ref[...] * bmat
    qacc[...] += jax.lax.dot_general(
        smt, qsb, (((0,), (0,)), ((), ())),
        preferred_element_type=jnp.float32)
    for cc in range(C):
        v = smt * bmat[:, cc:cc + 1]
        featacc[cc * G:(cc + 1) * G, :] += jax.lax.dot_general(
            v, x, (((0,), (0,)), ((), ())),
            preferred_element_type=jnp.float32)

    @pl.when(i == NB - 1)
    def _():
        z = jax.lax.dot_general(
            snfacc[...], wb1_ref[...], (((1,), (0,)), ((), ())),
            preferred_element_type=jnp.float32) + bb1_ref[...]
        z = jnp.maximum(z, 0.0)
        qv = jax.lax.dot_general(
            z, wb2_ref[...], (((1,), (0,)), ((), ())),
            preferred_element_type=jnp.float32) + bb2_ref[...]
        q_ref[...] = qacc[...] + qv
        feat_ref[...] = featacc[...]


def _final(agg2p, hself2, b2r, x, qs_r, nt_r, gid_r, wb1, bb1r, wb2, bb2r):
    return pl.pallas_call(
        _final_body,
        grid=(NB,),
        in_specs=[
            pl.BlockSpec((2, BN, TW), lambda i: (0, i, 0)),
            pl.BlockSpec((BN, TW), lambda i: (R * NB + i, 0)),
            pl.BlockSpec((1, C), lambda i: (0, 0)),
            pl.BlockSpec((BN, D), lambda i: (i, 0)),
            pl.BlockSpec((BN, 1), lambda i: (i, 0)),
            pl.BlockSpec((BN, 1), lambda i: (i, 0)),
            pl.BlockSpec((BN, 1), lambda i: (i, 0)),
            pl.BlockSpec((D, H), lambda i: (0, 0)),
            pl.BlockSpec((1, H), lambda i: (0, 0)),
            pl.BlockSpec((H, C), lambda i: (0, 0)),
            pl.BlockSpec((1, C), lambda i: (0, 0)),
        ],
        out_specs=[
            pl.BlockSpec((BN, C), lambda i: (i, 0)),
            pl.BlockSpec((G, C), lambda i: (0, 0)),
            pl.BlockSpec((C * G, D), lambda i: (0, 0)),
        ],
        out_shape=[
            jax.ShapeDtypeStruct((N, C), jnp.float32),
            jax.ShapeDtypeStruct((G, C), jnp.float32),
            jax.ShapeDtypeStruct((C * G, D), jnp.float32),
        ],
        scratch_shapes=[
            pltpu.VMEM((C * G, D), jnp.float32),
            pltpu.VMEM((G, C), jnp.float32),
            pltpu.VMEM((G, D), jnp.float32),
        ],
    )(agg2p, hself2, b2r, x, qs_r, nt_r, gid_r, wb1, bb1r, wb2, bb2r)


def kernel(node_feature, qs, edge_index, edge_type, node_type, graph_ids,
           W_rel1, W_self1, b1, W_rel2, W_self2, b2, Wb1, bb1, Wb2, bb2):
    f32 = jnp.float32
    src = edge_index[0]
    dst = edge_index[1]

    cidx = edge_type * N + src
    pad = EPAD - E
    ar = jnp.arange(pad, dtype=jnp.int32)
    cidx2 = jnp.concatenate([cidx, ar % (R * N)])
    cidx2 = cidx2.reshape(NW * NCH, CHUNK)
    dst2 = jnp.concatenate([dst, N + (ar % 16)])
    dst2 = dst2.reshape(NW * NCH, CHUNK)

    zeros_w = jnp.zeros((CHUNK, TW), f32)

    w1all = jnp.zeros((4, D, TW), f32)
    w1all = w1all.at[:R, :, :H].set(W_rel1)
    w1all = w1all.at[R, :, :H].set(W_self1)
    table1p = _dense1(node_feature, w1all)

    agg1p = _sc_edge_pass(table1p, cidx2, dst2, zeros_w)

    w2all = jnp.zeros((4, H, TW), f32)
    w2all = w2all.at[:R, :, :C].set(W_rel2)
    w2all = w2all.at[R, :, :C].set(W_self2)
    table2p = _dense2(agg1p, table1p, b1.reshape(1, H), w2all)

    agg2p = _sc_edge_pass(table2p, cidx2, dst2, zeros_w)

    ws, q_aggregated, featflat = _final(
        agg2p, table2p, b2.reshape(1, C), node_feature,
        qs.reshape(N, 1), node_type.reshape(N, 1), graph_ids.reshape(N, 1),
        Wb1, bb1.reshape(1, H), Wb2, bb2.reshape(1, C))

    feat = featflat.reshape(C, G, D).transpose(1, 0, 2)
    return (q_aggregated, ws, feat)


# speedup vs baseline: 3.0475x; 1.0202x over previous
import functools

import jax
import jax.numpy as jnp
from jax import lax
from jax.experimental import pallas as pl
from jax.experimental.pallas import tpu as pltpu
from jax.experimental.pallas import tpu_sc as plsc

N = 10000
E = 320000
D = 128
H = 64
C = 4
R = 3
G = 32

NW = 32
CHUNK = 128
NCH = 80
EPAD = NW * NCH * CHUNK
NACC = N + 16
TW = 72
DEGCOL = H

BN = 5000
NB = N // BN


def _dense1_body(x_ref, w_ref, y_ref):
    y = jax.lax.dot_general(
        x_ref[...], w_ref[0], (((1,), (0,)), ((), ())),
        preferred_element_type=jnp.float32)
    r = pl.program_id(0)
    col = jax.lax.broadcasted_iota(jnp.int32, y.shape, 1)
    y = jnp.where((col == DEGCOL) & (r < R), 1.0, y)
    y_ref[...] = y


def _dense1(x, w1all):
    return pl.pallas_call(
        _dense1_body,
        grid=(4, NB),
        in_specs=[
            pl.BlockSpec((BN, D), lambda r, i: (i, 0)),
            pl.BlockSpec((1, D, TW), lambda r, i: (r, 0, 0)),
        ],
        out_specs=pl.BlockSpec((BN, TW), lambda r, i: (r * NB + i, 0)),
        out_shape=jax.ShapeDtypeStruct((4 * N, TW), jnp.float32),
    )(x, w1all)


def _sc_edge_pass(table, cidx2, dst2, zeros_w):
    mesh = plsc.VectorSubcoreMesh(core_axis_name="c", subcore_axis_name="s")

    @functools.partial(
        pl.kernel,
        mesh=mesh,
        compiler_params=pltpu.CompilerParams(use_tc_tiling_on_sc=False),
        out_type=jax.ShapeDtypeStruct((2, N, TW), jnp.float32),
        scratch_types=[
            pltpu.VMEM((NCH // 2, CHUNK), jnp.int32),
            pltpu.VMEM((NCH // 2, CHUNK), jnp.int32),
            pltpu.VMEM((CHUNK, TW), jnp.float32),
            pltpu.VMEM((CHUNK, TW), jnp.float32),
            pltpu.SemaphoreType.DMA,
            pltpu.SemaphoreType.DMA,
            pltpu.SemaphoreType.DMA,
            pltpu.SemaphoreType.DMA,
            pltpu.VMEM_SHARED((NACC, TW), jnp.float32),
        ],
    )
    def k(table_hbm, cidx_hbm, dst_hbm, zw_hbm, agg_hbm,
          cidx_v, dst_v, rows_v, rows2_v, sem0, sem1, sem2, sem3, acc_sh):
        c = lax.axis_index("c")
        s = lax.axis_index("s")
        w = c * 16 + s
        pltpu.sync_copy(zw_hbm, rows_v)

        @pl.when(s < 15)
        def _():
            @pl.loop(0, 5)
            def _(jz):
                pltpu.sync_copy(rows_v, acc_sh.at[pl.ds(s * 640 + jz * 128,
                                                        128)])

        @pl.when(s == 15)
        def _():
            @pl.loop(0, 3)
            def _(jz):
                pltpu.sync_copy(rows_v, acc_sh.at[pl.ds(9600 + jz * 128,
                                                        128)])
            pltpu.sync_copy(rows_v.at[pl.ds(0, 32)],
                            acc_sh.at[pl.ds(9984, 32)])

        plsc.subcore_barrier()

        HH = NCH // 2
        for h in range(2):
            pltpu.sync_copy(cidx_hbm.at[pl.ds(w * NCH + h * HH, HH)], cidx_v)
            pltpu.sync_copy(dst_hbm.at[pl.ds(w * NCH + h * HH, HH)], dst_v)

            pltpu.make_async_copy(
                table_hbm.at[cidx_v.at[0]], rows_v, sem0).start()
            pltpu.make_async_copy(
                table_hbm.at[cidx_v.at[1]], rows2_v, sem1).start()

            @pl.loop(0, HH, step=2)
            def _(j):
                pltpu.make_async_copy(
                    table_hbm.at[cidx_v.at[0]], rows_v, sem0).wait()
                pltpu.make_async_copy(
                    rows_v, acc_sh.at[dst_v.at[j]], sem2).start(add=True)

                pltpu.make_async_copy(
                    table_hbm.at[cidx_v.at[1]], rows2_v, sem1).wait()
                pltpu.make_async_copy(
                    rows2_v, acc_sh.at[dst_v.at[j + 1]], sem3).start(add=True)

                pltpu.make_async_copy(
                    rows_v, acc_sh.at[dst_v.at[0]], sem2).wait()

                @pl.when(j + 2 < HH)
                def _():
                    pltpu.make_async_copy(
                        table_hbm.at[cidx_v.at[j + 2]], rows_v, sem0).start()

                pltpu.make_async_copy(
                    rows2_v, acc_sh.at[dst_v.at[0]], sem3).wait()

                @pl.when(j + 3 < HH)
                def _():
                    pltpu.make_async_copy(
                        table_hbm.at[cidx_v.at[j + 3]], rows2_v, sem1).start()

        plsc.subcore_barrier()

        @pl.when(s < 15)
        def _():
            @pl.loop(0, 5)
            def _(jz):
                pltpu.sync_copy(acc_sh.at[pl.ds(s * 640 + jz * 128, 128)],
                                rows_v)
                pltpu.sync_copy(rows_v,
                                agg_hbm.at[c, pl.ds(s * 640 + jz * 128, 128)])

        @pl.when(s == 15)
        def _():
            @pl.loop(0, 3)
            def _(jz):
                pltpu.sync_copy(acc_sh.at[pl.ds(9600 + jz * 128, 128)],
                                rows_v)
                pltpu.sync_copy(rows_v,
                                agg_hbm.at[c, pl.ds(9600 + jz * 128, 128)])
            pltpu.sync_copy(acc_sh.at[pl.ds(9984, 16)],
                            rows_v.at[pl.ds(0, 16)])
            pltpu.sync_copy(rows_v.at[pl.ds(0, 16)],
                            agg_hbm.at[c, pl.ds(9984, 16)])

    return k(table, cidx2, dst2, zeros_w)


def _dense2_body(agg_ref, xs_ref, b1_ref, w_ref, y_ref):
    agg = agg_ref[0] + agg_ref[1]
    deg = jnp.maximum(agg[:, DEGCOL:DEGCOL + 1], 1.0)
    h = agg[:, 0:H] / deg + xs_ref[...][:, 0:H] + b1_ref[...]
    h = jnp.maximum(h, 0.0)
    y = jax.lax.dot_general(
        h, w_ref[0], (((1,), (0,)), ((), ())),
        preferred_element_type=jnp.float32)
    r = pl.program_id(0)
    col = jax.lax.broadcasted_iota(jnp.int32, y.shape, 1)
    y = jnp.where((col == DEGCOL) & (r < R), 1.0, y)
    y_ref[...] = y


def _dense2(aggp, xself, b1r, w2all):
    return pl.pallas_call(
        _dense2_body,
        grid=(4, NB),
        in_specs=[
            pl.BlockSpec((2, BN, TW), lambda r, i: (0, i, 0)),
            pl.BlockSpec((BN, TW), lambda r, i: (R * NB + i, 0)),
            pl.BlockSpec((1, H), lambda r, i: (0, 0)),
            pl.BlockSpec((1, H, TW), lambda r, i: (r, 0, 0)),
        ],
        out_specs=pl.BlockSpec((BN, TW), lambda r, i: (r * NB + i, 0)),
        out_shape=jax.ShapeDtypeStruct((4 * N, TW), jnp.float32),
    )(aggp, xself, b1r, w2all)


def _final_body(agg2_ref, hs_ref, b2_ref, x_ref, qs_ref, nt_ref,
                gid_ref, wb1_ref, bb1_ref, wb2_ref, bb2_ref,
                ws_ref, q_ref, feat_ref,
                featacc, qacc, snfacc):
    i = pl.program_id(0)

    @pl.when(i == 0)
    def _():
        featacc[...] = jnp.zeros_like(featacc)
        qacc[...] = jnp.zeros_like(qacc)
        snfacc[...] = jnp.zeros_like(snfacc)

    agg2 = agg2_ref[0] + agg2_ref[1]
    deg = jnp.maximum(agg2[:, DEGCOL:DEGCOL + 1], 1.0)
    ws = agg2[:, 0:C] / deg + hs_ref[...][:, 0:C] + b2_ref[...]
    ws_ref[...] = ws

    mask = (nt_ref[...] == 0).astype(jnp.float32)
    bmat = ws * mask
    gid = gid_ref[...]
    iota = jax.lax.broadcasted_iota(jnp.int32, (1, G), 1)
    smt = (gid == iota).astype(jnp.float32)
    x = x_ref[...]

    snfacc[...] += jax.lax.dot_general(
        smt, x, (((0,), (0,)), ((), ())),
        preferred_element_type=jnp.float32)
    qsb = qs_ref[...] * bmat
    qacc[...] += jax.lax.dot_general(
        smt, qsb, (((0,), (0,)), ((), ())),
        preferred_element_type=jnp.float32)
    for cc in range(C):
        v = smt * bmat[:, cc:cc + 1]
        featacc[cc * G:(cc + 1) * G, :] += jax.lax.dot_general(
            v, x, (((0,), (0,)), ((), ())),
            preferred_element_type=jnp.float32)

    @pl.when(i == NB - 1)
    def _():
        z = jax.lax.dot_general(
            snfacc[...], wb1_ref[...], (((1,), (0,)), ((), ())),
            preferred_element_type=jnp.float32) + bb1_ref[...]
        z = jnp.maximum(z, 0.0)
        qv = jax.lax.dot_general(
            z, wb2_ref[...], (((1,), (0,)), ((), ())),
            preferred_element_type=jnp.float32) + bb2_ref[...]
        q_ref[...] = qacc[...] + qv
        feat_ref[...] = featacc[...]


def _final(agg2p, hself2, b2r, x, qs_r, nt_r, gid_r, wb1, bb1r, wb2, bb2r):
    return pl.pallas_call(
        _final_body,
        grid=(NB,),
        in_specs=[
            pl.BlockSpec((2, BN, TW), lambda i: (0, i, 0)),
            pl.BlockSpec((BN, TW), lambda i: (R * NB + i, 0)),
            pl.BlockSpec((1, C), lambda i: (0, 0)),
            pl.BlockSpec((BN, D), lambda i: (i, 0)),
            pl.BlockSpec((BN, 1), lambda i: (i, 0)),
            pl.BlockSpec((BN, 1), lambda i: (i, 0)),
            pl.BlockSpec((BN, 1), lambda i: (i, 0)),
            pl.BlockSpec((D, H), lambda i: (0, 0)),
            pl.BlockSpec((1, H), lambda i: (0, 0)),
            pl.BlockSpec((H, C), lambda i: (0, 0)),
            pl.BlockSpec((1, C), lambda i: (0, 0)),
        ],
        out_specs=[
            pl.BlockSpec((BN, C), lambda i: (i, 0)),
            pl.BlockSpec((G, C), lambda i: (0, 0)),
            pl.BlockSpec((C * G, D), lambda i: (0, 0)),
        ],
        out_shape=[
            jax.ShapeDtypeStruct((N, C), jnp.float32),
            jax.ShapeDtypeStruct((G, C), jnp.float32),
            jax.ShapeDtypeStruct((C * G, D), jnp.float32),
        ],
        scratch_shapes=[
            pltpu.VMEM((C * G, D), jnp.float32),
            pltpu.VMEM((G, C), jnp.float32),
            pltpu.VMEM((G, D), jnp.float32),
        ],
    )(agg2p, hself2, b2r, x, qs_r, nt_r, gid_r, wb1, bb1r, wb2, bb2r)


def kernel(node_feature, qs, edge_index, edge_type, node_type, graph_ids,
           W_rel1, W_self1, b1, W_rel2, W_self2, b2, Wb1, bb1, Wb2, bb2):
    f32 = jnp.float32
    src = edge_index[0]
    dst = edge_index[1]

    cidx = edge_type * N + src
    pad = EPAD - E
    ar = jnp.arange(pad, dtype=jnp.int32)
    cidx2 = jnp.concatenate([cidx, ar % (R * N)])
    cidx2 = cidx2.reshape(NW * NCH, CHUNK)
    dst2 = jnp.concatenate([dst, N + (ar % 16)])
    dst2 = dst2.reshape(NW * NCH, CHUNK)

    zeros_w = jnp.zeros((CHUNK, TW), f32)

    w1all = jnp.zeros((4, D, TW), f32)
    w1all = w1all.at[:R, :, :H].set(W_rel1)
    w1all = w1all.at[R, :, :H].set(W_self1)
    table1p = _dense1(node_feature, w1all)

    agg1p = _sc_edge_pass(table1p, cidx2, dst2, zeros_w)

    w2all = jnp.zeros((4, H, TW), f32)
    w2all = w2all.at[:R, :, :C].set(W_rel2)
    w2all = w2all.at[R, :, :C].set(W_self2)
    table2p = _dense2(agg1p, table1p, b1.reshape(1, H), w2all)

    agg2p = _sc_edge_pass(table2p, cidx2, dst2, zeros_w)

    ws, q_aggregated, featflat = _final(
        agg2p, table2p, b2.reshape(1, C), node_feature,
        qs.reshape(N, 1), node_type.reshape(N, 1), graph_ids.reshape(N, 1),
        Wb1, bb1.reshape(1, H), Wb2, bb2.reshape(1, C))

    feat = featflat.reshape(C, G, D).transpose(1, 0, 2)
    return (q_aggregated, ws, feat)


# speedup vs baseline: 3.5194x; 1.1549x over previous
import functools

import jax
import jax.numpy as jnp
from jax import lax
from jax.experimental import pallas as pl
from jax.experimental.pallas import tpu as pltpu
from jax.experimental.pallas import tpu_sc as plsc

N = 10000
E = 320000
D = 128
H = 64
C = 4
R = 3
G = 32

NW = 32
CHUNK = 128
NCH = 80
EPAD = NW * NCH * CHUNK
NACC = N + 16
TW = 72
DEGCOL = H

BN = 5000
NB = N // BN


def _dense1_body(x_ref, w_ref, y_ref):
    y = jax.lax.dot_general(
        x_ref[...], w_ref[0], (((1,), (0,)), ((), ())),
        preferred_element_type=jnp.float32)
    r = pl.program_id(0)
    col = jax.lax.broadcasted_iota(jnp.int32, y.shape, 1)
    y = jnp.where((col == DEGCOL) & (r < R), 1.0, y)
    y_ref[...] = y


def _dense1(x, w1all):
    return pl.pallas_call(
        _dense1_body,
        grid=(4, NB),
        in_specs=[
            pl.BlockSpec((BN, D), lambda r, i: (i, 0)),
            pl.BlockSpec((1, D, TW), lambda r, i: (r, 0, 0)),
        ],
        out_specs=pl.BlockSpec((BN, TW), lambda r, i: (r * NB + i, 0)),
        out_shape=jax.ShapeDtypeStruct((4 * N, TW), jnp.float32),
    )(x, w1all)


def _sc_edge_pass(table, cidx2, dst2, zeros_w):
    mesh = plsc.VectorSubcoreMesh(core_axis_name="c", subcore_axis_name="s")

    @functools.partial(
        pl.kernel,
        mesh=mesh,
        compiler_params=pltpu.CompilerParams(use_tc_tiling_on_sc=False),
        out_type=jax.ShapeDtypeStruct((2, N, TW), jnp.float32),
        scratch_types=[
            pltpu.VMEM((NCH // 2, CHUNK), jnp.int32),
            pltpu.VMEM((NCH // 2, CHUNK), jnp.int32),
            pltpu.VMEM((CHUNK, TW), jnp.float32),
            pltpu.VMEM((CHUNK, TW), jnp.float32),
            pltpu.VMEM((CHUNK, TW), jnp.float32),
            pltpu.VMEM((CHUNK, TW), jnp.float32),
            pltpu.SemaphoreType.DMA,
            pltpu.SemaphoreType.DMA,
            pltpu.SemaphoreType.DMA,
            pltpu.SemaphoreType.DMA,
            pltpu.SemaphoreType.DMA,
            pltpu.SemaphoreType.DMA,
            pltpu.SemaphoreType.DMA,
            pltpu.SemaphoreType.DMA,
            pltpu.VMEM_SHARED((NACC, TW), jnp.float32),
        ],
    )
    def k(table_hbm, cidx_hbm, dst_hbm, zw_hbm, agg_hbm,
          cidx_v, dst_v, rows_v, rows2_v, rows3_v, rows4_v,
          sem0, sem1, sem2, sem3, sem4, sem5, sem6, sem7, acc_sh):
        c = lax.axis_index("c")
        s = lax.axis_index("s")
        w = c * 16 + s
        pltpu.sync_copy(zw_hbm, rows_v)

        @pl.when(s < 15)
        def _():
            @pl.loop(0, 5)
            def _(jz):
                pltpu.sync_copy(rows_v, acc_sh.at[pl.ds(s * 640 + jz * 128,
                                                        128)])

        @pl.when(s == 15)
        def _():
            @pl.loop(0, 3)
            def _(jz):
                pltpu.sync_copy(rows_v, acc_sh.at[pl.ds(9600 + jz * 128,
                                                        128)])
            pltpu.sync_copy(rows_v.at[pl.ds(0, 32)],
                            acc_sh.at[pl.ds(9984, 32)])

        plsc.subcore_barrier()

        HH = NCH // 2
        bufs = [rows_v, rows2_v, rows3_v, rows4_v]
        gsems = [sem0, sem1, sem2, sem3]
        ssems = [sem4, sem5, sem6, sem7]
        for h in range(2):
            pltpu.sync_copy(cidx_hbm.at[pl.ds(w * NCH + h * HH, HH)], cidx_v)
            pltpu.sync_copy(dst_hbm.at[pl.ds(w * NCH + h * HH, HH)], dst_v)

            for kk in range(4):
                pltpu.make_async_copy(
                    table_hbm.at[cidx_v.at[kk]], bufs[kk], gsems[kk]).start()

            @pl.loop(0, HH, step=4)
            def _(j):
                for kk in range(4):
                    pltpu.make_async_copy(
                        table_hbm.at[cidx_v.at[0]], bufs[kk],
                        gsems[kk]).wait()
                    pltpu.make_async_copy(
                        bufs[kk], acc_sh.at[dst_v.at[j + kk]],
                        ssems[kk]).start(add=True)
                for kk in range(4):
                    pltpu.make_async_copy(
                        bufs[kk], acc_sh.at[dst_v.at[0]], ssems[kk]).wait()

                    @pl.when(j + kk + 4 < HH)
                    def _(kk=kk):
                        pltpu.make_async_copy(
                            table_hbm.at[cidx_v.at[j + kk + 4]], bufs[kk],
                            gsems[kk]).start()

        plsc.subcore_barrier()

        @pl.when(s < 15)
        def _():
            @pl.loop(0, 5)
            def _(jz):
                pltpu.sync_copy(acc_sh.at[pl.ds(s * 640 + jz * 128, 128)],
                                rows_v)
                pltpu.sync_copy(rows_v,
                                agg_hbm.at[c, pl.ds(s * 640 + jz * 128, 128)])

        @pl.when(s == 15)
        def _():
            @pl.loop(0, 3)
            def _(jz):
                pltpu.sync_copy(acc_sh.at[pl.ds(9600 + jz * 128, 128)],
                                rows_v)
                pltpu.sync_copy(rows_v,
                                agg_hbm.at[c, pl.ds(9600 + jz * 128, 128)])
            pltpu.sync_copy(acc_sh.at[pl.ds(9984, 16)],
                            rows_v.at[pl.ds(0, 16)])
            pltpu.sync_copy(rows_v.at[pl.ds(0, 16)],
                            agg_hbm.at[c, pl.ds(9984, 16)])

    return k(table, cidx2, dst2, zeros_w)


def _dense2_body(agg_ref, xs_ref, b1_ref, w_ref, y_ref):
    agg = agg_ref[0] + agg_ref[1]
    deg = jnp.maximum(agg[:, DEGCOL:DEGCOL + 1], 1.0)
    h = agg[:, 0:H] / deg + xs_ref[...][:, 0:H] + b1_ref[...]
    h = jnp.maximum(h, 0.0)
    y = jax.lax.dot_general(
        h, w_ref[0], (((1,), (0,)), ((), ())),
        preferred_element_type=jnp.float32)
    r = pl.program_id(0)
    col = jax.lax.broadcasted_iota(jnp.int32, y.shape, 1)
    y = jnp.where((col == DEGCOL) & (r < R), 1.0, y)
    y_ref[...] = y


def _dense2(aggp, xself, b1r, w2all):
    return pl.pallas_call(
        _dense2_body,
        grid=(4, NB),
        in_specs=[
            pl.BlockSpec((2, BN, TW), lambda r, i: (0, i, 0)),
            pl.BlockSpec((BN, TW), lambda r, i: (R * NB + i, 0)),
            pl.BlockSpec((1, H), lambda r, i: (0, 0)),
            pl.BlockSpec((1, H, TW), lambda r, i: (r, 0, 0)),
        ],
        out_specs=pl.BlockSpec((BN, TW), lambda r, i: (r * NB + i, 0)),
        out_shape=jax.ShapeDtypeStruct((4 * N, TW), jnp.float32),
    )(aggp, xself, b1r, w2all)


def _final_body(agg2_ref, hs_ref, b2_ref, x_ref, qs_ref, nt_ref,
                gid_ref, wb1_ref, bb1_ref, wb2_ref, bb2_ref,
                ws_ref, q_ref, feat_ref,
                featacc, qacc, snfacc):
    i = pl.program_id(0)

    @pl.when(i == 0)
    def _():
        featacc[...] = jnp.zeros_like(featacc)
        qacc[...] = jnp.zeros_like(qacc)
        snfacc[...] = jnp.zeros_like(snfacc)

    agg2 = agg2_ref[0] + agg2_ref[1]
    deg = jnp.maximum(agg2[:, DEGCOL:DEGCOL + 1], 1.0)
    ws = agg2[:, 0:C] / deg + hs_ref[...][:, 0:C] + b2_ref[...]
    ws_ref[...] = ws

    mask = (nt_ref[...] == 0).astype(jnp.float32)
    bmat = ws * mask
    gid = gid_ref[...]
    iota = jax.lax.broadcasted_iota(jnp.int32, (1, G), 1)
    smt = (gid == iota).astype(jnp.float32)
    x = x_ref[...]

    snfacc[...] += jax.lax.dot_general(
        smt, x, (((0,), (0,)), ((), ())),
        preferred_element_type=jnp.float32)
    qsb = qs_ref[...] * bmat
    qacc[...] += jax.lax.dot_general(
        smt, qsb, (((0,), (0,)), ((), ())),
        preferred_element_type=jnp.float32)
    for cc in range(C):
        v = smt * bmat[:, cc:cc + 1]
        featacc[cc * G:(cc + 1) * G, :] += jax.lax.dot_general(
            v, x, (((0,), (0,)), ((), ())),
            preferred_element_type=jnp.float32)

    @pl.when(i == NB - 1)
    def _():
        z = jax.lax.dot_general(
            snfacc[...], wb1_ref[...], (((1,), (0,)), ((), ())),
            preferred_element_type=jnp.float32) + bb1_ref[...]
        z = jnp.maximum(z, 0.0)
        qv = jax.lax.dot_general(
            z, wb2_ref[...], (((1,), (0,)), ((), ())),
            preferred_element_type=jnp.float32) + bb2_ref[...]
        q_ref[...] = qacc[...] + qv
        feat_ref[...] = featacc[...]


def _final(agg2p, hself2, b2r, x, qs_r, nt_r, gid_r, wb1, bb1r, wb2, bb2r):
    return pl.pallas_call(
        _final_body,
        grid=(NB,),
        in_specs=[
            pl.BlockSpec((2, BN, TW), lambda i: (0, i, 0)),
            pl.BlockSpec((BN, TW), lambda i: (R * NB + i, 0)),
            pl.BlockSpec((1, C), lambda i: (0, 0)),
            pl.BlockSpec((BN, D), lambda i: (i, 0)),
            pl.BlockSpec((BN, 1), lambda i: (i, 0)),
            pl.BlockSpec((BN, 1), lambda i: (i, 0)),
            pl.BlockSpec((BN, 1), lambda i: (i, 0)),
            pl.BlockSpec((D, H), lambda i: (0, 0)),
            pl.BlockSpec((1, H), lambda i: (0, 0)),
            pl.BlockSpec((H, C), lambda i: (0, 0)),
            pl.BlockSpec((1, C), lambda i: (0, 0)),
        ],
        out_specs=[
            pl.BlockSpec((BN, C), lambda i: (i, 0)),
            pl.BlockSpec((G, C), lambda i: (0, 0)),
            pl.BlockSpec((C * G, D), lambda i: (0, 0)),
        ],
        out_shape=[
            jax.ShapeDtypeStruct((N, C), jnp.float32),
            jax.ShapeDtypeStruct((G, C), jnp.float32),
            jax.ShapeDtypeStruct((C * G, D), jnp.float32),
        ],
        scratch_shapes=[
            pltpu.VMEM((C * G, D), jnp.float32),
            pltpu.VMEM((G, C), jnp.float32),
            pltpu.VMEM((G, D), jnp.float32),
        ],
    )(agg2p, hself2, b2r, x, qs_r, nt_r, gid_r, wb1, bb1r, wb2, bb2r)


def kernel(node_feature, qs, edge_index, edge_type, node_type, graph_ids,
           W_rel1, W_self1, b1, W_rel2, W_self2, b2, Wb1, bb1, Wb2, bb2):
    f32 = jnp.float32
    src = edge_index[0]
    dst = edge_index[1]

    cidx = edge_type * N + src
    pad = EPAD - E
    ar = jnp.arange(pad, dtype=jnp.int32)
    cidx2 = jnp.concatenate([cidx, ar % (R * N)])
    cidx2 = cidx2.reshape(NW * NCH, CHUNK)
    dst2 = jnp.concatenate([dst, N + (ar % 16)])
    dst2 = dst2.reshape(NW * NCH, CHUNK)

    zeros_w = jnp.zeros((CHUNK, TW), f32)

    w1all = jnp.zeros((4, D, TW), f32)
    w1all = w1all.at[:R, :, :H].set(W_rel1)
    w1all = w1all.at[R, :, :H].set(W_self1)
    table1p = _dense1(node_feature, w1all)

    agg1p = _sc_edge_pass(table1p, cidx2, dst2, zeros_w)

    w2all = jnp.zeros((4, H, TW), f32)
    w2all = w2all.at[:R, :, :C].set(W_rel2)
    w2all = w2all.at[R, :, :C].set(W_self2)
    table2p = _dense2(agg1p, table1p, b1.reshape(1, H), w2all)

    agg2p = _sc_edge_pass(table2p, cidx2, dst2, zeros_w)

    ws, q_aggregated, featflat = _final(
        agg2p, table2p, b2.reshape(1, C), node_feature,
        qs.reshape(N, 1), node_type.reshape(N, 1), graph_ids.reshape(N, 1),
        Wb1, bb1.reshape(1, H), Wb2, bb2.reshape(1, C))

    feat = featflat.reshape(C, G, D).transpose(1, 0, 2)
    return (q_aggregated, ws, feat)


# speedup vs baseline: 3.7917x; 1.0774x over previous
import functools

import jax
import jax.numpy as jnp
from jax import lax
from jax.experimental import pallas as pl
from jax.experimental.pallas import tpu as pltpu
from jax.experimental.pallas import tpu_sc as plsc

N = 10000
E = 320000
D = 128
H = 64
C = 4
R = 3
G = 32

NW = 32
CHUNK = 128
NCH = 80
EPAD = NW * NCH * CHUNK
NACC = N + 16
TW = 72
DEGCOL = H

BN = 5000
NB = N // BN


def _dense1_body(x_ref, w_ref, y_ref):
    y = jax.lax.dot_general(
        x_ref[...], w_ref[0], (((1,), (0,)), ((), ())),
        preferred_element_type=jnp.float32)
    r = pl.program_id(0)
    col = jax.lax.broadcasted_iota(jnp.int32, y.shape, 1)
    y = jnp.where((col == DEGCOL) & (r < R), 1.0, y)
    y_ref[...] = y


def _dense1(x, w1all):
    return pl.pallas_call(
        _dense1_body,
        grid=(4, NB),
        in_specs=[
            pl.BlockSpec((BN, D), lambda r, i: (i, 0)),
            pl.BlockSpec((1, D, TW), lambda r, i: (r, 0, 0)),
        ],
        out_specs=pl.BlockSpec((BN, TW), lambda r, i: (r * NB + i, 0)),
        out_shape=jax.ShapeDtypeStruct((4 * N, TW), jnp.float32),
    )(x, w1all)


def _sc_edge_pass(table, cidx2, dst2, zeros_w):
    mesh = plsc.VectorSubcoreMesh(core_axis_name="c", subcore_axis_name="s")

    @functools.partial(
        pl.kernel,
        mesh=mesh,
        compiler_params=pltpu.CompilerParams(use_tc_tiling_on_sc=False),
        out_type=jax.ShapeDtypeStruct((2, N, 128), jnp.float32),
        scratch_types=[
            pltpu.VMEM((NCH // 2, CHUNK), jnp.int32),
            pltpu.VMEM((NCH // 2, CHUNK), jnp.int32),
            pltpu.VMEM((CHUNK, TW), jnp.float32),
            pltpu.VMEM((CHUNK, TW), jnp.float32),
            pltpu.VMEM((CHUNK, TW), jnp.float32),
            pltpu.VMEM((CHUNK, TW), jnp.float32),
            pltpu.SemaphoreType.DMA,
            pltpu.SemaphoreType.DMA,
            pltpu.SemaphoreType.DMA,
            pltpu.SemaphoreType.DMA,
            pltpu.SemaphoreType.DMA,
            pltpu.SemaphoreType.DMA,
            pltpu.SemaphoreType.DMA,
            pltpu.SemaphoreType.DMA,
            pltpu.VMEM_SHARED((NACC, TW), jnp.float32),
        ],
    )
    def k(table_hbm, cidx_hbm, dst_hbm, zw_hbm, agg_hbm,
          cidx_v, dst_v, rows_v, rows2_v, rows3_v, rows4_v,
          sem0, sem1, sem2, sem3, sem4, sem5, sem6, sem7, acc_sh):
        c = lax.axis_index("c")
        s = lax.axis_index("s")
        w = c * 16 + s
        pltpu.sync_copy(zw_hbm, rows_v)

        @pl.when(s < 15)
        def _():
            @pl.loop(0, 5)
            def _(jz):
                pltpu.sync_copy(rows_v, acc_sh.at[pl.ds(s * 640 + jz * 128,
                                                        128)])

        @pl.when(s == 15)
        def _():
            @pl.loop(0, 3)
            def _(jz):
                pltpu.sync_copy(rows_v, acc_sh.at[pl.ds(9600 + jz * 128,
                                                        128)])
            pltpu.sync_copy(rows_v.at[pl.ds(0, 32)],
                            acc_sh.at[pl.ds(9984, 32)])

        plsc.subcore_barrier()

        HH = NCH // 2
        bufs = [rows_v, rows2_v, rows3_v, rows4_v]
        gsems = [sem0, sem1, sem2, sem3]
        ssems = [sem4, sem5, sem6, sem7]
        for h in range(2):
            pltpu.sync_copy(cidx_hbm.at[pl.ds(w * NCH + h * HH, HH)], cidx_v)
            pltpu.sync_copy(dst_hbm.at[pl.ds(w * NCH + h * HH, HH)], dst_v)

            for kk in range(4):
                pltpu.make_async_copy(
                    table_hbm.at[cidx_v.at[kk]], bufs[kk], gsems[kk]).start()

            @pl.loop(0, HH, step=4)
            def _(j):
                for kk in range(4):
                    pltpu.make_async_copy(
                        table_hbm.at[cidx_v.at[0]], bufs[kk],
                        gsems[kk]).wait()
                    pltpu.make_async_copy(
                        bufs[kk], acc_sh.at[dst_v.at[j + kk]],
                        ssems[kk]).start(add=True)
                for kk in range(4):
                    pltpu.make_async_copy(
                        bufs[kk], acc_sh.at[dst_v.at[0]], ssems[kk]).wait()

                    @pl.when(j + kk + 4 < HH)
                    def _(kk=kk):
                        pltpu.make_async_copy(
                            table_hbm.at[cidx_v.at[j + kk + 4]], bufs[kk],
                            gsems[kk]).start()

        plsc.subcore_barrier()

        @pl.when(s < 15)
        def _():
            @pl.loop(0, 5)
            def _(jz):
                pltpu.sync_copy(acc_sh.at[pl.ds(s * 640 + jz * 128, 128)],
                                rows_v)
                pltpu.sync_copy(rows_v,
                                agg_hbm.at[c, pl.ds(s * 640 + jz * 128, 128),
                                           pl.ds(0, TW)])

        @pl.when(s == 15)
        def _():
            @pl.loop(0, 3)
            def _(jz):
                pltpu.sync_copy(acc_sh.at[pl.ds(9600 + jz * 128, 128)],
                                rows_v)
                pltpu.sync_copy(rows_v,
                                agg_hbm.at[c, pl.ds(9600 + jz * 128, 128),
                                           pl.ds(0, TW)])
            pltpu.sync_copy(acc_sh.at[pl.ds(9984, 16)],
                            rows_v.at[pl.ds(0, 16)])
            pltpu.sync_copy(rows_v.at[pl.ds(0, 16)],
                            agg_hbm.at[c, pl.ds(9984, 16), pl.ds(0, TW)])

    return k(table, cidx2, dst2, zeros_w)


def _dense2_body(agg_ref, xs_ref, b1_ref, w_ref, y_ref):
    agg = agg_ref[0] + agg_ref[1]
    deg = jnp.maximum(agg[:, DEGCOL:DEGCOL + 1], 1.0)
    h = agg[:, 0:H] / deg + xs_ref[...][:, 0:H] + b1_ref[...]
    h = jnp.maximum(h, 0.0)
    y = jax.lax.dot_general(
        h, w_ref[0], (((1,), (0,)), ((), ())),
        preferred_element_type=jnp.float32)
    r = pl.program_id(0)
    col = jax.lax.broadcasted_iota(jnp.int32, y.shape, 1)
    y = jnp.where((col == DEGCOL) & (r < R), 1.0, y)
    y_ref[...] = y


def _dense2(aggp, xself, b1r, w2all):
    return pl.pallas_call(
        _dense2_body,
        grid=(4, NB),
        in_specs=[
            pl.BlockSpec((2, BN, 128), lambda r, i: (0, i, 0)),
            pl.BlockSpec((BN, TW), lambda r, i: (R * NB + i, 0)),
            pl.BlockSpec((1, H), lambda r, i: (0, 0)),
            pl.BlockSpec((1, H, TW), lambda r, i: (r, 0, 0)),
        ],
        out_specs=pl.BlockSpec((BN, TW), lambda r, i: (r * NB + i, 0)),
        out_shape=jax.ShapeDtypeStruct((4 * N, TW), jnp.float32),
    )(aggp, xself, b1r, w2all)


def _final_body(agg2_ref, hs_ref, b2_ref, x_ref, qs_ref, nt_ref,
                gid_ref, wb1_ref, bb1_ref, wb2_ref, bb2_ref,
                ws_ref, q_ref, feat_ref,
                featacc, qacc, snfacc):
    i = pl.program_id(0)

    @pl.when(i == 0)
    def _():
        featacc[...] = jnp.zeros_like(featacc)
        qacc[...] = jnp.zeros_like(qacc)
        snfacc[...] = jnp.zeros_like(snfacc)

    agg2 = agg2_ref[0] + agg2_ref[1]
    deg = jnp.maximum(agg2[:, DEGCOL:DEGCOL + 1], 1.0)
    ws = agg2[:, 0:C] / deg + hs_ref[...][:, 0:C] + b2_ref[...]
    ws_ref[...] = ws

    mask = (nt_ref[...] == 0).astype(jnp.float32)
    bmat = ws * mask
    gid = gid_ref[...]
    iota = jax.lax.broadcasted_iota(jnp.int32, (1, G), 1)
    smt = (gid == iota).astype(jnp.float32)
    x = x_ref[...]

    snfacc[...] += jax.lax.dot_general(
        smt, x, (((0,), (0,)), ((), ())),
        preferred_element_type=jnp.float32)
    qsb = qs_ref[...] * bmat
    qacc[...] += jax.lax.dot_general(
        smt, qsb, (((0,), (0,)), ((), ())),
        preferred_element_type=jnp.float32)
    for cc in range(C):
        v = smt * bmat[:, cc:cc + 1]
        featacc[cc * G:(cc + 1) * G, :] += jax.lax.dot_general(
            v, x, (((0,), (0,)), ((), ())),
            preferred_element_type=jnp.float32)

    @pl.when(i == NB - 1)
    def _():
        z = jax.lax.dot_general(
            snfacc[...], wb1_ref[...], (((1,), (0,)), ((), ())),
            preferred_element_type=jnp.float32) + bb1_ref[...]
        z = jnp.maximum(z, 0.0)
        qv = jax.lax.dot_general(
            z, wb2_ref[...], (((1,), (0,)), ((), ())),
            preferred_element_type=jnp.float32) + bb2_ref[...]
        q_ref[...] = qacc[...] + qv
        feat_ref[...] = featacc[...]


def _final(agg2p, hself2, b2r, x, qs_r, nt_r, gid_r, wb1, bb1r, wb2, bb2r):
    return pl.pallas_call(
        _final_body,
        grid=(NB,),
        in_specs=[
            pl.BlockSpec((2, BN, 128), lambda i: (0, i, 0)),
            pl.BlockSpec((BN, TW), lambda i: (R * NB + i, 0)),
            pl.BlockSpec((1, C), lambda i: (0, 0)),
            pl.BlockSpec((BN, D), lambda i: (i, 0)),
            pl.BlockSpec((BN, 1), lambda i: (i, 0)),
            pl.BlockSpec((BN, 1), lambda i: (i, 0)),
            pl.BlockSpec((BN, 1), lambda i: (i, 0)),
            pl.BlockSpec((D, H), lambda i: (0, 0)),
            pl.BlockSpec((1, H), lambda i: (0, 0)),
            pl.BlockSpec((H, C), lambda i: (0, 0)),
            pl.BlockSpec((1, C), lambda i: (0, 0)),
        ],
        out_specs=[
            pl.BlockSpec((BN, C), lambda i: (i, 0)),
            pl.BlockSpec((G, C), lambda i: (0, 0)),
            pl.BlockSpec((C * G, D), lambda i: (0, 0)),
        ],
        out_shape=[
            jax.ShapeDtypeStruct((N, C), jnp.float32),
            jax.ShapeDtypeStruct((G, C), jnp.float32),
            jax.ShapeDtypeStruct((C * G, D), jnp.float32),
        ],
        scratch_shapes=[
            pltpu.VMEM((C * G, D), jnp.float32),
            pltpu.VMEM((G, C), jnp.float32),
            pltpu.VMEM((G, D), jnp.float32),
        ],
    )(agg2p, hself2, b2r, x, qs_r, nt_r, gid_r, wb1, bb1r, wb2, bb2r)


def kernel(node_feature, qs, edge_index, edge_type, node_type, graph_ids,
           W_rel1, W_self1, b1, W_rel2, W_self2, b2, Wb1, bb1, Wb2, bb2):
    f32 = jnp.float32
    src = edge_index[0]
    dst = edge_index[1]

    cidx = edge_type * N + src
    pad = EPAD - E
    ar = jnp.arange(pad, dtype=jnp.int32)
    cidx2 = jnp.concatenate([cidx, ar % (R * N)])
    cidx2 = cidx2.reshape(NW * NCH, CHUNK)
    dst2 = jnp.concatenate([dst, N + (ar % 16)])
    dst2 = dst2.reshape(NW * NCH, CHUNK)

    zeros_w = jnp.zeros((CHUNK, TW), f32)

    w1all = jnp.zeros((4, D, TW), f32)
    w1all = w1all.at[:R, :, :H].set(W_rel1)
    w1all = w1all.at[R, :, :H].set(W_self1)
    table1p = _dense1(node_feature, w1all)

    agg1p = _sc_edge_pass(table1p, cidx2, dst2, zeros_w)

    w2all = jnp.zeros((4, H, TW), f32)
    w2all = w2all.at[:R, :, :C].set(W_rel2)
    w2all = w2all.at[R, :, :C].set(W_self2)
    table2p = _dense2(agg1p, table1p, b1.reshape(1, H), w2all)

    agg2p = _sc_edge_pass(table2p, cidx2, dst2, zeros_w)

    ws, q_aggregated, featflat = _final(
        agg2p, table2p, b2.reshape(1, C), node_feature,
        qs.reshape(N, 1), node_type.reshape(N, 1), graph_ids.reshape(N, 1),
        Wb1, bb1.reshape(1, H), Wb2, bb2.reshape(1, C))

    feat = featflat.reshape(C, G, D).transpose(1, 0, 2)
    return (q_aggregated, ws, feat)


# speedup vs baseline: 3.8000x; 1.0022x over previous
import functools

import jax
import jax.numpy as jnp
from jax import lax
from jax.experimental import pallas as pl
from jax.experimental.pallas import tpu as pltpu
from jax.experimental.pallas import tpu_sc as plsc

N = 10000
E = 320000
D = 128
H = 64
C = 4
R = 3
G = 32

NW = 32
CHUNK = 128
NCH = 80
EPAD = NW * NCH * CHUNK
NACC = N + 16
TW = 72
DEGCOL = H

BN = 5000
NB = N // BN


def _dense1_body(x_ref, w_ref, y_ref):
    y = jax.lax.dot_general(
        x_ref[...], w_ref[0], (((1,), (0,)), ((), ())),
        preferred_element_type=jnp.float32)
    r = pl.program_id(0)
    col = jax.lax.broadcasted_iota(jnp.int32, y.shape, 1)
    y = jnp.where((col == DEGCOL) & (r < R), 1.0, y)
    y_ref[...] = y


def _dense1(x, w1all):
    return pl.pallas_call(
        _dense1_body,
        grid=(4, NB),
        in_specs=[
            pl.BlockSpec((BN, D), lambda r, i: (i, 0)),
            pl.BlockSpec((1, D, TW), lambda r, i: (r, 0, 0)),
        ],
        out_specs=pl.BlockSpec((BN, TW), lambda r, i: (r * NB + i, 0)),
        out_shape=jax.ShapeDtypeStruct((4 * N, TW), jnp.float32),
    )(x, w1all)


def _sc_edge_pass(table, cidx2, dst2, zeros_w):
    mesh = plsc.VectorSubcoreMesh(core_axis_name="c", subcore_axis_name="s")

    @functools.partial(
        pl.kernel,
        mesh=mesh,
        compiler_params=pltpu.CompilerParams(use_tc_tiling_on_sc=False),
        out_type=jax.ShapeDtypeStruct((2, N, 128), jnp.float32),
        scratch_types=[
            pltpu.VMEM((NCH // 2, CHUNK), jnp.int32),
            pltpu.VMEM((NCH // 2, CHUNK), jnp.int32),
            pltpu.VMEM((CHUNK, TW), jnp.float32),
            pltpu.VMEM((CHUNK, TW), jnp.float32),
            pltpu.VMEM((CHUNK, TW), jnp.float32),
            pltpu.VMEM((CHUNK, TW), jnp.float32),
            pltpu.SemaphoreType.DMA,
            pltpu.SemaphoreType.DMA,
            pltpu.SemaphoreType.DMA,
            pltpu.SemaphoreType.DMA,
            pltpu.SemaphoreType.DMA,
            pltpu.SemaphoreType.DMA,
            pltpu.SemaphoreType.DMA,
            pltpu.SemaphoreType.DMA,
            pltpu.VMEM_SHARED((NACC, TW), jnp.float32),
        ],
    )
    def k(table_hbm, cidx_hbm, dst_hbm, zw_hbm, agg_hbm,
          cidx_v, dst_v, rows_v, rows2_v, rows3_v, rows4_v,
          sem0, sem1, sem2, sem3, sem4, sem5, sem6, sem7, acc_sh):
        c = lax.axis_index("c")
        s = lax.axis_index("s")
        w = c * 16 + s
        pltpu.sync_copy(zw_hbm, rows_v)

        @pl.when(s < 15)
        def _():
            @pl.loop(0, 5)
            def _(jz):
                pltpu.sync_copy(rows_v, acc_sh.at[pl.ds(s * 640 + jz * 128,
                                                        128)])

        @pl.when(s == 15)
        def _():
            @pl.loop(0, 3)
            def _(jz):
                pltpu.sync_copy(rows_v, acc_sh.at[pl.ds(9600 + jz * 128,
                                                        128)])
            pltpu.sync_copy(rows_v.at[pl.ds(0, 32)],
                            acc_sh.at[pl.ds(9984, 32)])

        plsc.subcore_barrier()

        HH = NCH // 2
        bufs = [rows_v, rows2_v, rows3_v, rows4_v]
        gsems = [sem0, sem1, sem2, sem3]
        ssems = [sem4, sem5, sem6, sem7]
        for h in range(2):
            pltpu.sync_copy(cidx_hbm.at[pl.ds(w * NCH + h * HH, HH)], cidx_v)
            pltpu.sync_copy(dst_hbm.at[pl.ds(w * NCH + h * HH, HH)], dst_v)

            for kk in range(4):
                pltpu.make_async_copy(
                    table_hbm.at[cidx_v.at[kk]], bufs[kk], gsems[kk]).start()

            @pl.loop(0, HH, step=4)
            def _(j):
                for kk in range(4):
                    pltpu.make_async_copy(
                        table_hbm.at[cidx_v.at[0]], bufs[kk],
                        gsems[kk]).wait()
                    pltpu.make_async_copy(
                        bufs[kk], acc_sh.at[dst_v.at[j + kk]],
                        ssems[kk]).start(add=True)
                for kk in range(4):
                    pltpu.make_async_copy(
                        bufs[kk], acc_sh.at[dst_v.at[0]], ssems[kk]).wait()

                    @pl.when(j + kk + 4 < HH)
                    def _(kk=kk):
                        pltpu.make_async_copy(
                            table_hbm.at[cidx_v.at[j + kk + 4]], bufs[kk],
                            gsems[kk]).start()

        plsc.subcore_barrier()

        @pl.when(s < 15)
        def _():
            pltpu.sync_copy(acc_sh.at[pl.ds(s * 640, 640)],
                            agg_hbm.at[c, pl.ds(s * 640, 640), pl.ds(0, TW)])

        @pl.when(s == 15)
        def _():
            pltpu.sync_copy(acc_sh.at[pl.ds(9600, 400)],
                            agg_hbm.at[c, pl.ds(9600, 400), pl.ds(0, TW)])

    return k(table, cidx2, dst2, zeros_w)


def _dense2_body(agg_ref, xs_ref, b1_ref, w_ref, y_ref):
    agg = agg_ref[0] + agg_ref[1]
    deg = jnp.maximum(agg[:, DEGCOL:DEGCOL + 1], 1.0)
    h = agg[:, 0:H] / deg + xs_ref[...][:, 0:H] + b1_ref[...]
    h = jnp.maximum(h, 0.0)
    y = jax.lax.dot_general(
        h, w_ref[0], (((1,), (0,)), ((), ())),
        preferred_element_type=jnp.float32)
    r = pl.program_id(0)
    col = jax.lax.broadcasted_iota(jnp.int32, y.shape, 1)
    y = jnp.where((col == DEGCOL) & (r < R), 1.0, y)
    y_ref[...] = y


def _dense2(aggp, xself, b1r, w2all):
    return pl.pallas_call(
        _dense2_body,
        grid=(4, NB),
        in_specs=[
            pl.BlockSpec((2, BN, 128), lambda r, i: (0, i, 0)),
            pl.BlockSpec((BN, TW), lambda r, i: (R * NB + i, 0)),
            pl.BlockSpec((1, H), lambda r, i: (0, 0)),
            pl.BlockSpec((1, H, TW), lambda r, i: (r, 0, 0)),
        ],
        out_specs=pl.BlockSpec((BN, TW), lambda r, i: (r * NB + i, 0)),
        out_shape=jax.ShapeDtypeStruct((4 * N, TW), jnp.float32),
    )(aggp, xself, b1r, w2all)


def _final_body(agg2_ref, hs_ref, b2_ref, x_ref, qs_ref, nt_ref,
                gid_ref, wb1_ref, bb1_ref, wb2_ref, bb2_ref,
                ws_ref, q_ref, feat_ref,
                featacc, qacc, snfacc):
    i = pl.program_id(0)

    @pl.when(i == 0)
    def _():
        featacc[...] = jnp.zeros_like(featacc)
        qacc[...] = jnp.zeros_like(qacc)
        snfacc[...] = jnp.zeros_like(snfacc)

    agg2 = agg2_ref[0] + agg2_ref[1]
    deg = jnp.maximum(agg2[:, DEGCOL:DEGCOL + 1], 1.0)
    ws = agg2[:, 0:C] / deg + hs_ref[...][:, 0:C] + b2_ref[...]
    ws_ref[...] = ws

    mask = (nt_ref[...] == 0).astype(jnp.float32)
    bmat = ws * mask
    gid = gid_ref[...]
    iota = jax.lax.broadcasted_iota(jnp.int32, (1, G), 1)
    smt = (gid == iota).astype(jnp.float32)
    x = x_ref[...]

    snfacc[...] += jax.lax.dot_general(
        smt, x, (((0,), (0,)), ((), ())),
        preferred_element_type=jnp.float32)
    qsb = qs_ref[...] * bmat
    qacc[...] += jax.lax.dot_general(
        smt, qsb, (((0,), (0,)), ((), ())),
        preferred_element_type=jnp.float32)
    for cc in range(C):
        v = smt * bmat[:, cc:cc + 1]
        featacc[cc * G:(cc + 1) * G, :] += jax.lax.dot_general(
            v, x, (((0,), (0,)), ((), ())),
            preferred_element_type=jnp.float32)

    @pl.when(i == NB - 1)
    def _():
        z = jax.lax.dot_general(
            snfacc[...], wb1_ref[...], (((1,), (0,)), ((), ())),
            preferred_element_type=jnp.float32) + bb1_ref[...]
        z = jnp.maximum(z, 0.0)
        qv = jax.lax.dot_general(
            z, wb2_ref[...], (((1,), (0,)), ((), ())),
            preferred_element_type=jnp.float32) + bb2_ref[...]
        q_ref[...] = qacc[...] + qv
        feat_ref[...] = featacc[...]


def _final(agg2p, hself2, b2r, x, qs_r, nt_r, gid_r, wb1, bb1r, wb2, bb2r):
    return pl.pallas_call(
        _final_body,
        grid=(NB,),
        in_specs=[
            pl.BlockSpec((2, BN, 128), lambda i: (0, i, 0)),
            pl.BlockSpec((BN, TW), lambda i: (R * NB + i, 0)),
            pl.BlockSpec((1, C), lambda i: (0, 0)),
            pl.BlockSpec((BN, D), lambda i: (i, 0)),
            pl.BlockSpec((BN, 1), lambda i: (i, 0)),
            pl.BlockSpec((BN, 1), lambda i: (i, 0)),
            pl.BlockSpec((BN, 1), lambda i: (i, 0)),
            pl.BlockSpec((D, H), lambda i: (0, 0)),
            pl.BlockSpec((1, H), lambda i: (0, 0)),
            pl.BlockSpec((H, C), lambda i: (0, 0)),
            pl.BlockSpec((1, C), lambda i: (0, 0)),
        ],
        out_specs=[
            pl.BlockSpec((BN, C), lambda i: (i, 0)),
            pl.BlockSpec((G, C), lambda i: (0, 0)),
            pl.BlockSpec((C * G, D), lambda i: (0, 0)),
        ],
        out_shape=[
            jax.ShapeDtypeStruct((N, C), jnp.float32),
            jax.ShapeDtypeStruct((G, C), jnp.float32),
            jax.ShapeDtypeStruct((C * G, D), jnp.float32),
        ],
        scratch_shapes=[
            pltpu.VMEM((C * G, D), jnp.float32),
            pltpu.VMEM((G, C), jnp.float32),
            pltpu.VMEM((G, D), jnp.float32),
        ],
    )(agg2p, hself2, b2r, x, qs_r, nt_r, gid_r, wb1, bb1r, wb2, bb2r)


def kernel(node_feature, qs, edge_index, edge_type, node_type, graph_ids,
           W_rel1, W_self1, b1, W_rel2, W_self2, b2, Wb1, bb1, Wb2, bb2):
    f32 = jnp.float32
    src = edge_index[0]
    dst = edge_index[1]

    cidx = edge_type * N + src
    pad = EPAD - E
    ar = jnp.arange(pad, dtype=jnp.int32)
    cidx2 = jnp.concatenate([cidx, ar % (R * N)])
    cidx2 = cidx2.reshape(NW * NCH, CHUNK)
    dst2 = jnp.concatenate([dst, N + (ar % 16)])
    dst2 = dst2.reshape(NW * NCH, CHUNK)

    zeros_w = jnp.zeros((CHUNK, TW), f32)

    w1all = jnp.zeros((4, D, TW), f32)
    w1all = w1all.at[:R, :, :H].set(W_rel1)
    w1all = w1all.at[R, :, :H].set(W_self1)
    table1p = _dense1(node_feature, w1all)

    agg1p = _sc_edge_pass(table1p, cidx2, dst2, zeros_w)

    w2all = jnp.zeros((4, H, TW), f32)
    w2all = w2all.at[:R, :, :C].set(W_rel2)
    w2all = w2all.at[R, :, :C].set(W_self2)
    table2p = _dense2(agg1p, table1p, b1.reshape(1, H), w2all)

    agg2p = _sc_edge_pass(table2p, cidx2, dst2, zeros_w)

    ws, q_aggregated, featflat = _final(
        agg2p, table2p, b2.reshape(1, C), node_feature,
        qs.reshape(N, 1), node_type.reshape(N, 1), graph_ids.reshape(N, 1),
        Wb1, bb1.reshape(1, H), Wb2, bb2.reshape(1, C))

    feat = featflat.reshape(C, G, D).transpose(1, 0, 2)
    return (q_aggregated, ws, feat)


# speedup vs baseline: 3.9748x; 1.0460x over previous
import functools

import jax
import jax.numpy as jnp
from jax import lax
from jax.experimental import pallas as pl
from jax.experimental.pallas import tpu as pltpu
from jax.experimental.pallas import tpu_sc as plsc

N = 10000
E = 320000
D = 128
H = 64
C = 4
R = 3
G = 32

NW = 32
CHUNK = 128
NCH = 80
EPAD = NW * NCH * CHUNK
NACC = N + 16
TW = 72
DEGCOL = H

BN = 5000
NB = N // BN


def _dense1_body(x_ref, w_ref, y_ref):
    y = jax.lax.dot_general(
        x_ref[...], w_ref[0], (((1,), (0,)), ((), ())),
        preferred_element_type=jnp.float32)
    r = pl.program_id(1)
    col = jax.lax.broadcasted_iota(jnp.int32, y.shape, 1)
    y = jnp.where((col == DEGCOL) & (r < R), 1.0, y)
    y_ref[...] = y


def _dense1(x, w1all):
    return pl.pallas_call(
        _dense1_body,
        grid=(NB, 4),
        in_specs=[
            pl.BlockSpec((BN, D), lambda i, r: (i, 0)),
            pl.BlockSpec((1, D, TW), lambda i, r: (r, 0, 0)),
        ],
        out_specs=pl.BlockSpec((BN, TW), lambda i, r: (r * NB + i, 0)),
        out_shape=jax.ShapeDtypeStruct((4 * N, TW), jnp.float32),
    )(x, w1all)


def _sc_edge_pass(table, cidx2, dst2, zeros_w):
    mesh = plsc.VectorSubcoreMesh(core_axis_name="c", subcore_axis_name="s")

    @functools.partial(
        pl.kernel,
        mesh=mesh,
        compiler_params=pltpu.CompilerParams(use_tc_tiling_on_sc=False),
        out_type=jax.ShapeDtypeStruct((2, N, 128), jnp.float32),
        scratch_types=[
            pltpu.VMEM((NCH // 2, CHUNK), jnp.int32),
            pltpu.VMEM((NCH // 2, CHUNK), jnp.int32),
            pltpu.VMEM((CHUNK, TW), jnp.float32),
            pltpu.VMEM((CHUNK, TW), jnp.float32),
            pltpu.VMEM((CHUNK, TW), jnp.float32),
            pltpu.VMEM((CHUNK, TW), jnp.float32),
            pltpu.SemaphoreType.DMA,
            pltpu.SemaphoreType.DMA,
            pltpu.SemaphoreType.DMA,
            pltpu.SemaphoreType.DMA,
            pltpu.SemaphoreType.DMA,
            pltpu.SemaphoreType.DMA,
            pltpu.SemaphoreType.DMA,
            pltpu.SemaphoreType.DMA,
            pltpu.VMEM_SHARED((NACC, TW), jnp.float32),
        ],
    )
    def k(table_hbm, cidx_hbm, dst_hbm, zw_hbm, agg_hbm,
          cidx_v, dst_v, rows_v, rows2_v, rows3_v, rows4_v,
          sem0, sem1, sem2, sem3, sem4, sem5, sem6, sem7, acc_sh):
        c = lax.axis_index("c")
        s = lax.axis_index("s")
        w = c * 16 + s
        pltpu.sync_copy(zw_hbm, rows_v)

        @pl.when(s < 15)
        def _():
            @pl.loop(0, 5)
            def _(jz):
                pltpu.sync_copy(rows_v, acc_sh.at[pl.ds(s * 640 + jz * 128,
                                                        128)])

        @pl.when(s == 15)
        def _():
            @pl.loop(0, 3)
            def _(jz):
                pltpu.sync_copy(rows_v, acc_sh.at[pl.ds(9600 + jz * 128,
                                                        128)])
            pltpu.sync_copy(rows_v.at[pl.ds(0, 32)],
                            acc_sh.at[pl.ds(9984, 32)])

        plsc.subcore_barrier()

        HH = NCH // 2
        bufs = [rows_v, rows2_v, rows3_v, rows4_v]
        gsems = [sem0, sem1, sem2, sem3]
        ssems = [sem4, sem5, sem6, sem7]
        for h in range(2):
            pltpu.sync_copy(cidx_hbm.at[pl.ds(w * NCH + h * HH, HH)], cidx_v)
            pltpu.sync_copy(dst_hbm.at[pl.ds(w * NCH + h * HH, HH)], dst_v)

            for kk in range(4):
                pltpu.make_async_copy(
                    table_hbm.at[cidx_v.at[kk]], bufs[kk], gsems[kk]).start()

            @pl.loop(0, HH, step=4)
            def _(j):
                for kk in range(4):
                    pltpu.make_async_copy(
                        table_hbm.at[cidx_v.at[0]], bufs[kk],
                        gsems[kk]).wait()
                    pltpu.make_async_copy(
                        bufs[kk], acc_sh.at[dst_v.at[j + kk]],
                        ssems[kk]).start(add=True)
                for kk in range(4):
                    pltpu.make_async_copy(
                        bufs[kk], acc_sh.at[dst_v.at[0]], ssems[kk]).wait()

                    @pl.when(j + kk + 4 < HH)
                    def _(kk=kk):
                        pltpu.make_async_copy(
                            table_hbm.at[cidx_v.at[j + kk + 4]], bufs[kk],
                            gsems[kk]).start()

        plsc.subcore_barrier()

        @pl.when(s < 15)
        def _():
            pltpu.sync_copy(acc_sh.at[pl.ds(s * 640, 640)],
                            agg_hbm.at[c, pl.ds(s * 640, 640), pl.ds(0, TW)])

        @pl.when(s == 15)
        def _():
            pltpu.sync_copy(acc_sh.at[pl.ds(9600, 400)],
                            agg_hbm.at[c, pl.ds(9600, 400), pl.ds(0, TW)])

    return k(table, cidx2, dst2, zeros_w)


def _dense2_body(agg_ref, xs_ref, b1_ref, w_ref, y_ref):
    agg = agg_ref[0] + agg_ref[1]
    deg = jnp.maximum(agg[:, DEGCOL:DEGCOL + 1], 1.0)
    h = agg[:, 0:H] / deg + xs_ref[...][:, 0:H] + b1_ref[...]
    h = jnp.maximum(h, 0.0)
    y = jax.lax.dot_general(
        h, w_ref[0], (((1,), (0,)), ((), ())),
        preferred_element_type=jnp.float32)
    r = pl.program_id(1)
    col = jax.lax.broadcasted_iota(jnp.int32, y.shape, 1)
    y = jnp.where((col == DEGCOL) & (r < R), 1.0, y)
    y_ref[...] = y


def _dense2(aggp, xself, b1r, w2all):
    return pl.pallas_call(
        _dense2_body,
        grid=(NB, 4),
        in_specs=[
            pl.BlockSpec((2, BN, 128), lambda i, r: (0, i, 0)),
            pl.BlockSpec((BN, TW), lambda i, r: (R * NB + i, 0)),
            pl.BlockSpec((1, H), lambda i, r: (0, 0)),
            pl.BlockSpec((1, H, TW), lambda i, r: (r, 0, 0)),
        ],
        out_specs=pl.BlockSpec((BN, TW), lambda i, r: (r * NB + i, 0)),
        out_shape=jax.ShapeDtypeStruct((4 * N, TW), jnp.float32),
    )(aggp, xself, b1r, w2all)


def _final_body(agg2_ref, hs_ref, b2_ref, x_ref, qs_ref, nt_ref,
                gid_ref, wb1_ref, bb1_ref, wb2_ref, bb2_ref,
                ws_ref, q_ref, feat_ref,
                featacc, qacc, snfacc):
    i = pl.program_id(0)

    @pl.when(i == 0)
    def _():
        featacc[...] = jnp.zeros_like(featacc)
        qacc[...] = jnp.zeros_like(qacc)
        snfacc[...] = jnp.zeros_like(snfacc)

    agg2 = agg2_ref[0] + agg2_ref[1]
    deg = jnp.maximum(agg2[:, DEGCOL:DEGCOL + 1], 1.0)
    ws = agg2[:, 0:C] / deg + hs_ref[...][:, 0:C] + b2_ref[...]
    ws_ref[...] = ws

    mask = (nt_ref[...] == 0).astype(jnp.float32)
    bmat = ws * mask
    gid = gid_ref[...]
    iota = jax.lax.broadcasted_iota(jnp.int32, (1, G), 1)
    smt = (gid == iota).astype(jnp.float32)
    x = x_ref[...]

    snfacc[...] += jax.lax.dot_general(
        smt, x, (((0,), (0,)), ((), ())),
        preferred_element_type=jnp.float32)
    qsb = qs_ref[...] * bmat
    qacc[...] += jax.lax.dot_general(
        smt, qsb, (((0,), (0,)), ((), ())),
        preferred_element_type=jnp.float32)
    for cc in range(C):
        v = smt * bmat[:, cc:cc + 1]
        featacc[cc * G:(cc + 1) * G, :] += jax.lax.dot_general(
            v, x, (((0,), (0,)), ((), ())),
            preferred_element_type=jnp.float32)

    @pl.when(i == NB - 1)
    def _():
        z = jax.lax.dot_general(
            snfacc[...], wb1_ref[...], (((1,), (0,)), ((), ())),
            preferred_element_type=jnp.float32) + bb1_ref[...]
        z = jnp.maximum(z, 0.0)
        qv = jax.lax.dot_general(
            z, wb2_ref[...], (((1,), (0,)), ((), ())),
            preferred_element_type=jnp.float32) + bb2_ref[...]
        q_ref[...] = qacc[...] + qv
        feat_ref[...] = featacc[...]


def _final(agg2p, hself2, b2r, x, qs_r, nt_r, gid_r, wb1, bb1r, wb2, bb2r):
    return pl.pallas_call(
        _final_body,
        grid=(NB,),
        in_specs=[
            pl.BlockSpec((2, BN, 128), lambda i: (0, i, 0)),
            pl.BlockSpec((BN, TW), lambda i: (R * NB + i, 0)),
            pl.BlockSpec((1, C), lambda i: (0, 0)),
            pl.BlockSpec((BN, D), lambda i: (i, 0)),
            pl.BlockSpec((BN, 1), lambda i: (i, 0)),
            pl.BlockSpec((BN, 1), lambda i: (i, 0)),
            pl.BlockSpec((BN, 1), lambda i: (i, 0)),
            pl.BlockSpec((D, H), lambda i: (0, 0)),
            pl.BlockSpec((1, H), lambda i: (0, 0)),
            pl.BlockSpec((H, C), lambda i: (0, 0)),
            pl.BlockSpec((1, C), lambda i: (0, 0)),
        ],
        out_specs=[
            pl.BlockSpec((BN, C), lambda i: (i, 0)),
            pl.BlockSpec((G, C), lambda i: (0, 0)),
            pl.BlockSpec((C * G, D), lambda i: (0, 0)),
        ],
        out_shape=[
            jax.ShapeDtypeStruct((N, C), jnp.float32),
            jax.ShapeDtypeStruct((G, C), jnp.float32),
            jax.ShapeDtypeStruct((C * G, D), jnp.float32),
        ],
        scratch_shapes=[
            pltpu.VMEM((C * G, D), jnp.float32),
            pltpu.VMEM((G, C), jnp.float32),
            pltpu.VMEM((G, D), jnp.float32),
        ],
    )(agg2p, hself2, b2r, x, qs_r, nt_r, gid_r, wb1, bb1r, wb2, bb2r)


def kernel(node_feature, qs, edge_index, edge_type, node_type, graph_ids,
           W_rel1, W_self1, b1, W_rel2, W_self2, b2, Wb1, bb1, Wb2, bb2):
    f32 = jnp.float32
    src = edge_index[0]
    dst = edge_index[1]

    cidx = edge_type * N + src
    pad = EPAD - E
    ar = jnp.arange(pad, dtype=jnp.int32)
    cidx2 = jnp.concatenate([cidx, ar % (R * N)])
    cidx2 = cidx2.reshape(NW * NCH, CHUNK)
    dst2 = jnp.concatenate([dst, N + (ar % 16)])
    dst2 = dst2.reshape(NW * NCH, CHUNK)

    zeros_w = jnp.zeros((CHUNK, TW), f32)

    w1all = jnp.zeros((4, D, TW), f32)
    w1all = w1all.at[:R, :, :H].set(W_rel1)
    w1all = w1all.at[R, :, :H].set(W_self1)
    table1p = _dense1(node_feature, w1all)

    agg1p = _sc_edge_pass(table1p, cidx2, dst2, zeros_w)

    w2all = jnp.zeros((4, H, TW), f32)
    w2all = w2all.at[:R, :, :C].set(W_rel2)
    w2all = w2all.at[R, :, :C].set(W_self2)
    table2p = _dense2(agg1p, table1p, b1.reshape(1, H), w2all)

    agg2p = _sc_edge_pass(table2p, cidx2, dst2, zeros_w)

    ws, q_aggregated, featflat = _final(
        agg2p, table2p, b2.reshape(1, C), node_feature,
        qs.reshape(N, 1), node_type.reshape(N, 1), graph_ids.reshape(N, 1),
        Wb1, bb1.reshape(1, H), Wb2, bb2.reshape(1, C))

    feat = featflat.reshape(C, G, D).transpose(1, 0, 2)
    return (q_aggregated, ws, feat)
